# baseline (speedup 1.0000x reference)
import dataclasses
import functools

import jax
import jax.numpy as jnp
from jax import lax
from jax.experimental import pallas as pl
from jax.experimental.pallas import tpu as pltpu
from jax.experimental.pallas import tpu_sc as plsc

B, T, D, E, K, H, OUT = 32, 196, 768, 16, 2, 128, 18
N = B * T
TILE = 128
NT = N // TILE
RCAP = (N * K // TILE + E) * TILE
NRT = RCAP // TILE
G = 64
NG = N // G
NSUB = 32
F32 = jnp.float32
I32 = jnp.int32

_SC_PARAMS = pltpu.CompilerParams()
if "needs_layout_passes" in pltpu.CompilerParams.__dataclass_fields__:
    _SC_PARAMS = dataclasses.replace(_SC_PARAMS, needs_layout_passes=False)


def _gate_body(x_ref, gw_ref, gb_ref, code_ref, w_ref,
               po_ref, eot_ref, c0_ref, c1_ref):
    i = pl.program_id(0)

    @pl.when(i == 0)
    def _():
        c0_ref[...] = jnp.zeros_like(c0_ref)
        c1_ref[...] = jnp.zeros_like(c1_ref)

    x = x_ref[...]
    logits = jnp.dot(x, gw_ref[...], preferred_element_type=F32)
    logits = logits + gb_ref[...]
    m = jnp.max(logits, axis=1, keepdims=True)
    ex = jnp.exp(logits - m)
    probs = ex / jnp.sum(ex, axis=1, keepdims=True)

    idx = lax.broadcasted_iota(I32, (TILE, E), 1)
    m0 = jnp.max(probs, axis=1, keepdims=True)
    e0 = jnp.min(jnp.where(probs >= m0, idx, E + 1), axis=1)
    oh0 = (idx == e0[:, None]).astype(F32)
    probs2 = jnp.where(oh0 > 0, -1.0, probs)
    m1 = jnp.max(probs2, axis=1, keepdims=True)
    e1 = jnp.min(jnp.where(probs2 >= m1, idx, E + 1), axis=1)
    oh1 = (idx == e1[:, None]).astype(F32)

    r = lax.broadcasted_iota(I32, (TILE, TILE), 0)
    c = lax.broadcasted_iota(I32, (TILE, TILE), 1)
    lt = (r > c).astype(jnp.bfloat16)
    ohs = jnp.concatenate([oh0, oh1], axis=1).astype(jnp.bfloat16)
    exs = jnp.dot(lt, ohs, preferred_element_type=F32)
    rank0 = jnp.sum(exs[:, :E] * oh0, axis=1) + jnp.sum(oh0 * c0_ref[...], axis=1)
    rank1 = jnp.sum(exs[:, E:] * oh1, axis=1) + jnp.sum(oh1 * c1_ref[...], axis=1)
    c0_ref[...] = c0_ref[...] + jnp.sum(oh0, axis=0, keepdims=True)
    c1_ref[...] = c1_ref[...] + jnp.sum(oh1, axis=0, keepdims=True)

    col = lax.broadcasted_iota(I32, (TILE, 2), 1)
    code0 = (rank0.astype(I32) << 4) | e0
    code1 = (rank1.astype(I32) << 4) | e1
    code_ref[...] = jnp.where(col == 0, code0[:, None], code1[:, None])
    w_ref[...] = jnp.where(col == 0, probs[:, 0:1], probs[:, 1:2])

    @pl.when(i == NT - 1)
    def _():
        h0 = c0_ref[...]
        ht = c0_ref[...] + c1_ref[...]
        padded = jnp.ceil(ht / TILE) * TILE
        rr = lax.broadcasted_iota(I32, (E, E), 0)
        cc = lax.broadcasted_iota(I32, (E, E), 1)
        su = (rr < cc).astype(F32)
        po = jnp.dot(padded, su, preferred_element_type=F32)
        prow = lax.broadcasted_iota(I32, (8, E), 0)
        po_ref[...] = jnp.where(prow == 1, po + h0, po).astype(I32)
        tpos = lax.broadcasted_iota(I32, (TILE, E), 0).astype(F32) * TILE
        eot = jnp.sum((tpos >= po).astype(I32), axis=1) - 1
        eot = jnp.clip(eot, 0, E - 1)
        eot_ref[...] = jnp.broadcast_to(eot[None, :], (8, TILE))


def _gate(x2d, gate_w, gate_b):
    return pl.pallas_call(
        _gate_body,
        grid=(NT,),
        in_specs=[
            pl.BlockSpec((TILE, D), lambda i: (i, 0)),
            pl.BlockSpec((D, E), lambda i: (0, 0)),
            pl.BlockSpec((1, E), lambda i: (0, 0)),
        ],
        out_specs=[
            pl.BlockSpec((TILE, 2), lambda i: (i, 0)),
            pl.BlockSpec((TILE, 2), lambda i: (i, 0)),
            pl.BlockSpec((8, E), lambda i: (0, 0)),
            pl.BlockSpec((8, TILE), lambda i: (0, 0)),
        ],
        out_shape=[
            jax.ShapeDtypeStruct((N, 2), I32),
            jax.ShapeDtypeStruct((N, 2), F32),
            jax.ShapeDtypeStruct((8, E), I32),
            jax.ShapeDtypeStruct((8, TILE), I32),
        ],
        scratch_shapes=[pltpu.VMEM((1, E), F32), pltpu.VMEM((1, E), F32)],
        compiler_params=pltpu.CompilerParams(
            dimension_semantics=("arbitrary",)),
    )(x2d, gate_w, gate_b.reshape(1, E))


DI = D // 2
def _dispatch(xb, code, po2):
    @functools.partial(
        pl.kernel,
        mesh=plsc.VectorSubcoreMesh(core_axis_name="c", subcore_axis_name="s"),
        out_type=[jax.ShapeDtypeStruct((RCAP, D), F32),
                  jax.ShapeDtypeStruct((2, N), I32)],
        scratch_types=[pltpu.VMEM((G, D), F32),
                       pltpu.VMEM((G, 2), I32),
                       pltpu.VMEM((2, G), I32),
                       pltpu.VMEM((2, E), I32)],
        compiler_params=_SC_PARAMS,
    )
    def s1(x_hbm, code_hbm, po_hbm, xs_hbm, dest_hbm,
           rows_v, cd_v, dscr_v, po_v):
        wid = lax.axis_index("s") * 2 + lax.axis_index("c")
        pltpu.sync_copy(po_hbm.at[pl.ds(0, 2), :], po_v)
        for j in range(4):
            g = wid + NSUB * j

            @pl.when(g < NG)
            def _():
                base = g * G
                pltpu.sync_copy(x_hbm.at[pl.ds(base, G), :], rows_v)
                pltpu.sync_copy(code_hbm.at[pl.ds(base, G), :], cd_v)
                for k in range(2):
                    kf = jnp.full((16,), k, I32)
                    for jj in range(G // 16):
                        ridx = lax.iota(I32, 16) + 16 * jj
                        cd = plsc.load_gather(cd_v, [ridx, kf])
                        e = jnp.bitwise_and(cd, E - 1)
                        rk = lax.shift_right_logical(cd, 4)
                        po = plsc.load_gather(po_v, [kf, e])
                        dscr_v[k, pl.ds(16 * jj, 16)] = po + rk
                for k in range(2):
                    pltpu.sync_copy(dscr_v.at[k], dest_hbm.at[k, pl.ds(base, G)])
                    pltpu.sync_copy(rows_v, xs_hbm.at[dscr_v.at[k]])

    return s1(xb, code, po2)


def _mlp_body(eot_ref, xs_ref, w1_ref, b1_ref, w2_ref, b2_ref, w3_ref, b3_ref,
              y_ref):
    x = xs_ref[...].astype(jnp.bfloat16)
    h = jnp.dot(x, w1_ref[0], preferred_element_type=F32) + b1_ref[0]
    h = jnp.maximum(h, 0.0).astype(jnp.bfloat16)
    h = jnp.dot(h, w2_ref[0], preferred_element_type=F32) + b2_ref[0]
    h = jnp.maximum(h, 0.0).astype(jnp.bfloat16)
    h = jnp.dot(h, w3_ref[0], preferred_element_type=F32) + b3_ref[0]
    y_ref[...] = jnp.maximum(h, 0.0)


def _moe_mlp(eot, xs, ew1, eb1, ew2, eb2, ew3, eb3):
    grid_spec = pltpu.PrefetchScalarGridSpec(
        num_scalar_prefetch=1,
        grid=(NRT,),
        in_specs=[
            pl.BlockSpec((TILE, D), lambda i, eot: (i, 0)),
            pl.BlockSpec((1, D, H), lambda i, eot: (eot[0, i], 0, 0)),
            pl.BlockSpec((1, 1, H), lambda i, eot: (eot[0, i], 0, 0)),
            pl.BlockSpec((1, H, H), lambda i, eot: (eot[0, i], 0, 0)),
            pl.BlockSpec((1, 1, H), lambda i, eot: (eot[0, i], 0, 0)),
            pl.BlockSpec((1, H, H), lambda i, eot: (eot[0, i], 0, 0)),
            pl.BlockSpec((1, 1, H), lambda i, eot: (eot[0, i], 0, 0)),
        ],
        out_specs=pl.BlockSpec((TILE, H), lambda i, eot: (i, 0)),
    )
    return pl.pallas_call(
        _mlp_body,
        grid_spec=grid_spec,
        out_shape=jax.ShapeDtypeStruct((RCAP, H), F32),
        compiler_params=pltpu.CompilerParams(
            dimension_semantics=("arbitrary",)),
    )(eot, xs, ew1.astype(jnp.bfloat16), eb1.reshape(E, 1, H),
      ew2.astype(jnp.bfloat16), eb2.reshape(E, 1, H),
      ew3.astype(jnp.bfloat16), eb3.reshape(E, 1, H))


def _gather_pairs(dest, y):
    @functools.partial(
        pl.kernel,
        mesh=plsc.VectorSubcoreMesh(core_axis_name="c", subcore_axis_name="s"),
        out_type=jax.ShapeDtypeStruct((2, N, H), F32),
        scratch_types=[pltpu.VMEM((2, G), I32), pltpu.VMEM((G, H), F32)],
        compiler_params=_SC_PARAMS,
    )
    def s2(dest_hbm, y_hbm, z_hbm, d_v, z_v):
        wid = lax.axis_index("s") * 2 + lax.axis_index("c")
        for j in range(4):
            g = wid + NSUB * j

            @pl.when(g < NG)
            def _():
                base = g * G
                for k in range(2):
                    pltpu.sync_copy(dest_hbm.at[k, pl.ds(base, G)], d_v.at[k])
                    pltpu.sync_copy(y_hbm.at[d_v.at[k]], z_v)
                    pltpu.sync_copy(z_v, z_hbm.at[k, pl.ds(base, G), :])

    return s2(dest, y)


def _combine_body(z_ref, w_ref, w2_ref, b2_ref, w3_ref, b3_ref, y_ref):
    w = w_ref[...]
    a = w[:, 0:1] * z_ref[0] + w[:, 1:2] * z_ref[1]
    a = jnp.maximum(a, 0.0).astype(jnp.bfloat16)
    y = jnp.dot(a, w2_ref[...], preferred_element_type=F32) + b2_ref[...]
    y = jnp.maximum(y, 0.0).astype(jnp.bfloat16)
    y = jnp.dot(y, w3_ref[...], preferred_element_type=F32) + b3_ref[...]
    y_ref[...] = jnp.maximum(y, 0.0).astype(jnp.bfloat16)


def _combine(z, wco, fc2_w, fc2_b, fc3_w, fc3_b):
    return pl.pallas_call(
        _combine_body,
        grid=(NT,),
        in_specs=[
            pl.BlockSpec((2, TILE, H), lambda i: (0, i, 0)),
            pl.BlockSpec((TILE, 2), lambda i: (i, 0)),
            pl.BlockSpec((H, H), lambda i: (0, 0)),
            pl.BlockSpec((1, H), lambda i: (0, 0)),
            pl.BlockSpec((H, H), lambda i: (0, 0)),
            pl.BlockSpec((1, H), lambda i: (0, 0)),
        ],
        out_specs=pl.BlockSpec((TILE, H), lambda i: (i, 0)),
        out_shape=jax.ShapeDtypeStruct((N, H), jnp.bfloat16),
        compiler_params=pltpu.CompilerParams(
            dimension_semantics=("arbitrary",)),
    )(z, wco, fc2_w.astype(jnp.bfloat16), fc2_b.reshape(1, H),
      fc3_w.astype(jnp.bfloat16), fc3_b.reshape(1, H))


TK = 28
NKC = T // TK


def _tail_body(y_ref, w4_ref, b4_ref, w5_ref, b5_ref, w6_ref, b6_ref,
               o_ref, acc_ref):
    i = pl.program_id(0)

    @pl.when(i == 0)
    def _():
        acc_ref[...] = jnp.zeros_like(acc_ref)

    acc = acc_ref[...]
    for tk in range(TK):
        acc += jnp.dot(y_ref[tk], w4_ref[tk].astype(jnp.bfloat16),
                       preferred_element_type=F32)
    acc_ref[...] = acc

    @pl.when(i == NKC - 1)
    def _():
        z = jnp.maximum(acc_ref[...] + b4_ref[...], 0.0).astype(jnp.bfloat16)
        z = jnp.dot(z, w5_ref[...].astype(jnp.bfloat16),
                    preferred_element_type=F32) + b5_ref[...]
        z = jnp.maximum(z, 0.0).astype(jnp.bfloat16)
        o_ref[...] = jnp.dot(z, w6_ref[...].astype(jnp.bfloat16),
                             preferred_element_type=F32) + b6_ref[...]


def _tail(y1t, fc4_w, fc4_b, fc5_w, fc5_b, fc6_w, fc6_b):
    return pl.pallas_call(
        _tail_body,
        grid=(NKC,),
        in_specs=[
            pl.BlockSpec((TK, B, H), lambda i: (i, 0, 0)),
            pl.BlockSpec((TK, H, H), lambda i: (i, 0, 0)),
            pl.BlockSpec((1, H), lambda i: (0, 0)),
            pl.BlockSpec((H, H), lambda i: (0, 0)),
            pl.BlockSpec((1, H), lambda i: (0, 0)),
            pl.BlockSpec((H, OUT), lambda i: (0, 0)),
            pl.BlockSpec((1, OUT), lambda i: (0, 0)),
        ],
        out_specs=pl.BlockSpec((B, OUT), lambda i: (0, 0)),
        out_shape=jax.ShapeDtypeStruct((B, OUT), F32),
        scratch_shapes=[pltpu.VMEM((B, H), F32)],
        compiler_params=pltpu.CompilerParams(
            dimension_semantics=("arbitrary",)),
    )(y1t, fc4_w.reshape(T, H, H), fc4_b.reshape(1, H),
      fc5_w, fc5_b.reshape(1, H), fc6_w, fc6_b.reshape(1, OUT))


def kernel(x, gate_w, gate_b, ew1, eb1, ew2, eb2, ew3, eb3,
           fc2_w, fc2_b, fc3_w, fc3_b, fc4_w, fc4_b,
           fc5_w, fc5_b, fc6_w, fc6_b):
    x2d = jnp.transpose(x, (1, 0, 2)).reshape(N, D)
    code, wco, po2, eot_pad = _gate(x2d, gate_w, gate_b)
    xs, dest = _dispatch(x2d, code, po2)
    y = _moe_mlp(eot_pad, xs, ew1, eb1, ew2, eb2, ew3, eb3)
    z = _gather_pairs(dest, y)
    y1 = _combine(z, wco, fc2_w, fc2_b, fc3_w, fc3_b)
    return _tail(y1.reshape(T, B, H), fc4_w, fc4_b, fc5_w, fc5_b,
                 fc6_w, fc6_b)


# speedup vs baseline: 1.1011x; 1.1011x over previous
import dataclasses
import functools

import jax
import jax.numpy as jnp
from jax import lax
from jax.experimental import pallas as pl
from jax.experimental.pallas import tpu as pltpu
from jax.experimental.pallas import tpu_sc as plsc

B, T, D, E, K, H, OUT = 32, 196, 768, 16, 2, 128, 18
N = B * T
TILE = 128
NT = N // TILE
RCAP = (N * K // TILE + E) * TILE
NRT = RCAP // TILE
G = 64
NG = N // G
NSUB = 32
F32 = jnp.float32
I32 = jnp.int32

_SC_PARAMS = pltpu.CompilerParams()
if "needs_layout_passes" in pltpu.CompilerParams.__dataclass_fields__:
    _SC_PARAMS = dataclasses.replace(_SC_PARAMS, needs_layout_passes=False)


def _gate_body(x_ref, gw_ref, gb_ref, code_ref, w_ref,
               po_ref, eot_ref, c0_ref, c1_ref):
    i = pl.program_id(0)

    @pl.when(i == 0)
    def _():
        c0_ref[...] = jnp.zeros_like(c0_ref)
        c1_ref[...] = jnp.zeros_like(c1_ref)

    x = x_ref[...]
    logits = jnp.dot(x, gw_ref[...], preferred_element_type=F32)
    logits = logits + gb_ref[...]
    m = jnp.max(logits, axis=1, keepdims=True)
    ex = jnp.exp(logits - m)
    probs = ex / jnp.sum(ex, axis=1, keepdims=True)

    idx = lax.broadcasted_iota(I32, (TILE, E), 1)
    m0 = jnp.max(probs, axis=1, keepdims=True)
    e0 = jnp.min(jnp.where(probs >= m0, idx, E + 1), axis=1)
    oh0 = (idx == e0[:, None]).astype(F32)
    probs2 = jnp.where(oh0 > 0, -1.0, probs)
    m1 = jnp.max(probs2, axis=1, keepdims=True)
    e1 = jnp.min(jnp.where(probs2 >= m1, idx, E + 1), axis=1)
    oh1 = (idx == e1[:, None]).astype(F32)

    r = lax.broadcasted_iota(I32, (TILE, TILE), 0)
    c = lax.broadcasted_iota(I32, (TILE, TILE), 1)
    lt = (r > c).astype(F32)
    ex0 = jnp.dot(lt, oh0, preferred_element_type=F32)
    ex1 = jnp.dot(lt, oh1, preferred_element_type=F32)
    rank0 = jnp.sum(ex0 * oh0, axis=1) + jnp.sum(oh0 * c0_ref[...], axis=1)
    rank1 = jnp.sum(ex1 * oh1, axis=1) + jnp.sum(oh1 * c1_ref[...], axis=1)
    c0_ref[...] = c0_ref[...] + jnp.sum(oh0, axis=0, keepdims=True)
    c1_ref[...] = c1_ref[...] + jnp.sum(oh1, axis=0, keepdims=True)

    col = lax.broadcasted_iota(I32, (TILE, 2), 1)
    code0 = (rank0.astype(I32) << 4) | e0
    code1 = (rank1.astype(I32) << 4) | e1
    code_ref[...] = jnp.where(col == 0, code0[:, None], code1[:, None])
    w_ref[...] = jnp.where(col == 0, probs[:, 0:1], probs[:, 1:2])

    @pl.when(i == NT - 1)
    def _():
        h0 = c0_ref[...]
        ht = c0_ref[...] + c1_ref[...]
        padded = jnp.ceil(ht / TILE) * TILE
        rr = lax.broadcasted_iota(I32, (E, E), 0)
        cc = lax.broadcasted_iota(I32, (E, E), 1)
        su = (rr < cc).astype(F32)
        po = jnp.dot(padded, su, preferred_element_type=F32)
        prow = lax.broadcasted_iota(I32, (8, E), 0)
        po_ref[...] = jnp.where(prow == 1, po + h0, po).astype(I32)
        tpos = lax.broadcasted_iota(I32, (TILE, E), 0).astype(F32) * TILE
        eot = jnp.sum((tpos >= po).astype(I32), axis=1) - 1
        eot = jnp.clip(eot, 0, E - 1)
        eot_ref[...] = jnp.broadcast_to(eot[None, :], (8, TILE))


def _gate(x2d, gate_w, gate_b):
    return pl.pallas_call(
        _gate_body,
        grid=(NT,),
        in_specs=[
            pl.BlockSpec((TILE, D), lambda i: (i, 0)),
            pl.BlockSpec((D, E), lambda i: (0, 0)),
            pl.BlockSpec((1, E), lambda i: (0, 0)),
        ],
        out_specs=[
            pl.BlockSpec((TILE, 2), lambda i: (i, 0)),
            pl.BlockSpec((TILE, 2), lambda i: (i, 0)),
            pl.BlockSpec((8, E), lambda i: (0, 0)),
            pl.BlockSpec((8, TILE), lambda i: (0, 0)),
        ],
        out_shape=[
            jax.ShapeDtypeStruct((N, 2), I32),
            jax.ShapeDtypeStruct((N, 2), F32),
            jax.ShapeDtypeStruct((8, E), I32),
            jax.ShapeDtypeStruct((8, TILE), I32),
        ],
        scratch_shapes=[pltpu.VMEM((1, E), F32), pltpu.VMEM((1, E), F32)],
        compiler_params=pltpu.CompilerParams(
            dimension_semantics=("arbitrary",)),
    )(x2d, gate_w, gate_b.reshape(1, E))


DI = D // 2
def _dispatch(xb, code, po2):
    @functools.partial(
        pl.kernel,
        mesh=plsc.VectorSubcoreMesh(core_axis_name="c", subcore_axis_name="s"),
        out_type=[jax.ShapeDtypeStruct((RCAP, D), F32),
                  jax.ShapeDtypeStruct((2, N), I32)],
        scratch_types=[pltpu.VMEM((G, D), F32),
                       pltpu.VMEM((G, 2), I32),
                       pltpu.VMEM((2, G), I32),
                       pltpu.VMEM((2, E), I32)],
        compiler_params=_SC_PARAMS,
    )
    def s1(x_hbm, code_hbm, po_hbm, xs_hbm, dest_hbm,
           rows_v, cd_v, dscr_v, po_v):
        wid = lax.axis_index("s") * 2 + lax.axis_index("c")
        pltpu.sync_copy(po_hbm.at[pl.ds(0, 2), :], po_v)
        for j in range(4):
            g = wid + NSUB * j

            @pl.when(g < NG)
            def _():
                base = g * G
                pltpu.sync_copy(x_hbm.at[pl.ds(base, G), :], rows_v)
                pltpu.sync_copy(code_hbm.at[pl.ds(base, G), :], cd_v)
                for k in range(2):
                    kf = jnp.full((16,), k, I32)
                    for jj in range(G // 16):
                        ridx = lax.iota(I32, 16) + 16 * jj
                        cd = plsc.load_gather(cd_v, [ridx, kf])
                        e = jnp.bitwise_and(cd, E - 1)
                        rk = lax.shift_right_logical(cd, 4)
                        po = plsc.load_gather(po_v, [kf, e])
                        dscr_v[k, pl.ds(16 * jj, 16)] = po + rk
                for k in range(2):
                    pltpu.sync_copy(dscr_v.at[k], dest_hbm.at[k, pl.ds(base, G)])
                    pltpu.sync_copy(rows_v, xs_hbm.at[dscr_v.at[k]])

    return s1(xb, code, po2)


def _mlp_body(eot_ref, xs_ref, w1_ref, b1_ref, w2_ref, b2_ref, w3_ref, b3_ref,
              y_ref):
    x = xs_ref[...].astype(jnp.bfloat16)
    h = jnp.dot(x, w1_ref[0], preferred_element_type=F32) + b1_ref[0]
    h = jnp.maximum(h, 0.0).astype(jnp.bfloat16)
    h = jnp.dot(h, w2_ref[0], preferred_element_type=F32) + b2_ref[0]
    h = jnp.maximum(h, 0.0).astype(jnp.bfloat16)
    h = jnp.dot(h, w3_ref[0], preferred_element_type=F32) + b3_ref[0]
    y_ref[...] = jnp.maximum(h, 0.0)


def _moe_mlp(eot, xs, ew1, eb1, ew2, eb2, ew3, eb3):
    grid_spec = pltpu.PrefetchScalarGridSpec(
        num_scalar_prefetch=1,
        grid=(NRT,),
        in_specs=[
            pl.BlockSpec((TILE, D), lambda i, eot: (i, 0)),
            pl.BlockSpec((1, D, H), lambda i, eot: (eot[0, i], 0, 0)),
            pl.BlockSpec((1, 1, H), lambda i, eot: (eot[0, i], 0, 0)),
            pl.BlockSpec((1, H, H), lambda i, eot: (eot[0, i], 0, 0)),
            pl.BlockSpec((1, 1, H), lambda i, eot: (eot[0, i], 0, 0)),
            pl.BlockSpec((1, H, H), lambda i, eot: (eot[0, i], 0, 0)),
            pl.BlockSpec((1, 1, H), lambda i, eot: (eot[0, i], 0, 0)),
        ],
        out_specs=pl.BlockSpec((TILE, H), lambda i, eot: (i, 0)),
    )
    return pl.pallas_call(
        _mlp_body,
        grid_spec=grid_spec,
        out_shape=jax.ShapeDtypeStruct((RCAP, H), F32),
        compiler_params=pltpu.CompilerParams(
            dimension_semantics=("arbitrary",)),
    )(eot, xs, ew1.astype(jnp.bfloat16), eb1.reshape(E, 1, H),
      ew2.astype(jnp.bfloat16), eb2.reshape(E, 1, H),
      ew3.astype(jnp.bfloat16), eb3.reshape(E, 1, H))


def _gather_pairs(dest, y):
    @functools.partial(
        pl.kernel,
        mesh=plsc.VectorSubcoreMesh(core_axis_name="c", subcore_axis_name="s"),
        out_type=jax.ShapeDtypeStruct((2, N, H), F32),
        scratch_types=[pltpu.VMEM((2, G), I32), pltpu.VMEM((G, H), F32)],
        compiler_params=_SC_PARAMS,
    )
    def s2(dest_hbm, y_hbm, z_hbm, d_v, z_v):
        wid = lax.axis_index("s") * 2 + lax.axis_index("c")
        for j in range(4):
            g = wid + NSUB * j

            @pl.when(g < NG)
            def _():
                base = g * G
                for k in range(2):
                    pltpu.sync_copy(dest_hbm.at[k, pl.ds(base, G)], d_v.at[k])
                    pltpu.sync_copy(y_hbm.at[d_v.at[k]], z_v)
                    pltpu.sync_copy(z_v, z_hbm.at[k, pl.ds(base, G), :])

    return s2(dest, y)


def _combine_body(z_ref, w_ref, w2_ref, b2_ref, w3_ref, b3_ref, y_ref):
    w = w_ref[...]
    a = w[:, 0:1] * z_ref[0] + w[:, 1:2] * z_ref[1]
    a = jnp.maximum(a, 0.0).astype(jnp.bfloat16)
    y = jnp.dot(a, w2_ref[...], preferred_element_type=F32) + b2_ref[...]
    y = jnp.maximum(y, 0.0).astype(jnp.bfloat16)
    y = jnp.dot(y, w3_ref[...], preferred_element_type=F32) + b3_ref[...]
    y_ref[...] = jnp.maximum(y, 0.0).astype(jnp.bfloat16)


def _combine(z, wco, fc2_w, fc2_b, fc3_w, fc3_b):
    return pl.pallas_call(
        _combine_body,
        grid=(NT,),
        in_specs=[
            pl.BlockSpec((2, TILE, H), lambda i: (0, i, 0)),
            pl.BlockSpec((TILE, 2), lambda i: (i, 0)),
            pl.BlockSpec((H, H), lambda i: (0, 0)),
            pl.BlockSpec((1, H), lambda i: (0, 0)),
            pl.BlockSpec((H, H), lambda i: (0, 0)),
            pl.BlockSpec((1, H), lambda i: (0, 0)),
        ],
        out_specs=pl.BlockSpec((TILE, H), lambda i: (i, 0)),
        out_shape=jax.ShapeDtypeStruct((N, H), jnp.bfloat16),
        compiler_params=pltpu.CompilerParams(
            dimension_semantics=("arbitrary",)),
    )(z, wco, fc2_w.astype(jnp.bfloat16), fc2_b.reshape(1, H),
      fc3_w.astype(jnp.bfloat16), fc3_b.reshape(1, H))


TK = 28
NKC = T // TK


def _tail_body(y_ref, w4_ref, b4_ref, w5_ref, b5_ref, w6_ref, b6_ref,
               o_ref, acc_ref):
    i = pl.program_id(0)

    @pl.when(i == 0)
    def _():
        acc_ref[...] = jnp.zeros_like(acc_ref)

    acc = acc_ref[...]
    for tk in range(TK):
        acc += jnp.dot(y_ref[tk], w4_ref[tk].astype(jnp.bfloat16),
                       preferred_element_type=F32)
    acc_ref[...] = acc

    @pl.when(i == NKC - 1)
    def _():
        z = jnp.maximum(acc_ref[...] + b4_ref[...], 0.0).astype(jnp.bfloat16)
        z = jnp.dot(z, w5_ref[...].astype(jnp.bfloat16),
                    preferred_element_type=F32) + b5_ref[...]
        z = jnp.maximum(z, 0.0).astype(jnp.bfloat16)
        o_ref[...] = jnp.dot(z, w6_ref[...].astype(jnp.bfloat16),
                             preferred_element_type=F32) + b6_ref[...]


def _tail(y1t, fc4_w, fc4_b, fc5_w, fc5_b, fc6_w, fc6_b):
    return pl.pallas_call(
        _tail_body,
        grid=(NKC,),
        in_specs=[
            pl.BlockSpec((TK, B, H), lambda i: (i, 0, 0)),
            pl.BlockSpec((TK, H, H), lambda i: (i, 0, 0)),
            pl.BlockSpec((1, H), lambda i: (0, 0)),
            pl.BlockSpec((H, H), lambda i: (0, 0)),
            pl.BlockSpec((1, H), lambda i: (0, 0)),
            pl.BlockSpec((H, OUT), lambda i: (0, 0)),
            pl.BlockSpec((1, OUT), lambda i: (0, 0)),
        ],
        out_specs=pl.BlockSpec((B, OUT), lambda i: (0, 0)),
        out_shape=jax.ShapeDtypeStruct((B, OUT), F32),
        scratch_shapes=[pltpu.VMEM((B, H), F32)],
        compiler_params=pltpu.CompilerParams(
            dimension_semantics=("arbitrary",)),
    )(y1t, fc4_w.reshape(T, H, H), fc4_b.reshape(1, H),
      fc5_w, fc5_b.reshape(1, H), fc6_w, fc6_b.reshape(1, OUT))


def kernel(x, gate_w, gate_b, ew1, eb1, ew2, eb2, ew3, eb3,
           fc2_w, fc2_b, fc3_w, fc3_b, fc4_w, fc4_b,
           fc5_w, fc5_b, fc6_w, fc6_b):
    x2d = jnp.transpose(x, (1, 0, 2)).reshape(N, D)
    code, wco, po2, eot_pad = _gate(x2d, gate_w, gate_b)
    xs, dest = _dispatch(x2d, code, po2)
    y = _moe_mlp(eot_pad, xs, ew1, eb1, ew2, eb2, ew3, eb3)
    z = _gather_pairs(dest, y)
    y1 = _combine(z, wco, fc2_w, fc2_b, fc3_w, fc3_b)
    return _tail(y1.reshape(T, B, H), fc4_w, fc4_b, fc5_w, fc5_b,
                 fc6_w, fc6_b)


# speedup vs baseline: 1.1470x; 1.0417x over previous
import dataclasses
import functools

import jax
import jax.numpy as jnp
from jax import lax
from jax.experimental import pallas as pl
from jax.experimental.pallas import tpu as pltpu
from jax.experimental.pallas import tpu_sc as plsc

B, T, D, E, K, H, OUT = 32, 196, 768, 16, 2, 128, 18
N = B * T
TILE = 128
NT = N // TILE
RCAP = (N * K // TILE + E) * TILE
NRT = RCAP // TILE
G = 64
NG = N // G
NSUB = 32
F32 = jnp.float32
I32 = jnp.int32

_SC_PARAMS = pltpu.CompilerParams()
if "needs_layout_passes" in pltpu.CompilerParams.__dataclass_fields__:
    _SC_PARAMS = dataclasses.replace(_SC_PARAMS, needs_layout_passes=False)


def _gate_body(x_ref, gw_ref, gb_ref, code_ref, w_ref,
               po_ref, eot_ref, c0_ref, c1_ref):
    i = pl.program_id(0)

    @pl.when(i == 0)
    def _():
        c0_ref[...] = jnp.zeros_like(c0_ref)
        c1_ref[...] = jnp.zeros_like(c1_ref)

    x = x_ref[...]
    logits = jnp.dot(x, gw_ref[...], preferred_element_type=F32)
    logits = logits + gb_ref[...]
    m = jnp.max(logits, axis=1, keepdims=True)
    ex = jnp.exp(logits - m)
    probs = ex / jnp.sum(ex, axis=1, keepdims=True)

    idx = lax.broadcasted_iota(I32, (TILE, E), 1)
    m0 = jnp.max(probs, axis=1, keepdims=True)
    e0 = jnp.min(jnp.where(probs >= m0, idx, E + 1), axis=1)
    oh0 = (idx == e0[:, None]).astype(F32)
    probs2 = jnp.where(oh0 > 0, -1.0, probs)
    m1 = jnp.max(probs2, axis=1, keepdims=True)
    e1 = jnp.min(jnp.where(probs2 >= m1, idx, E + 1), axis=1)
    oh1 = (idx == e1[:, None]).astype(F32)

    r = lax.broadcasted_iota(I32, (TILE, TILE), 0)
    c = lax.broadcasted_iota(I32, (TILE, TILE), 1)
    lt = (r > c).astype(F32)
    ex0 = jnp.dot(lt, oh0, preferred_element_type=F32)
    ex1 = jnp.dot(lt, oh1, preferred_element_type=F32)
    rank0 = jnp.sum(ex0 * oh0, axis=1) + jnp.sum(oh0 * c0_ref[...], axis=1)
    rank1 = jnp.sum(ex1 * oh1, axis=1) + jnp.sum(oh1 * c1_ref[...], axis=1)
    c0_ref[...] = c0_ref[...] + jnp.sum(oh0, axis=0, keepdims=True)
    c1_ref[...] = c1_ref[...] + jnp.sum(oh1, axis=0, keepdims=True)

    col = lax.broadcasted_iota(I32, (TILE, 2), 1)
    code0 = (rank0.astype(I32) << 4) | e0
    code1 = (rank1.astype(I32) << 4) | e1
    code_ref[...] = jnp.where(col == 0, code0[:, None], code1[:, None])
    w_ref[...] = jnp.where(col == 0, probs[:, 0:1], probs[:, 1:2])

    @pl.when(i == NT - 1)
    def _():
        h0 = c0_ref[...]
        ht = c0_ref[...] + c1_ref[...]
        padded = jnp.ceil(ht / TILE) * TILE
        rr = lax.broadcasted_iota(I32, (E, E), 0)
        cc = lax.broadcasted_iota(I32, (E, E), 1)
        su = (rr < cc).astype(F32)
        po = jnp.dot(padded, su, preferred_element_type=F32)
        prow = lax.broadcasted_iota(I32, (8, E), 0)
        po_ref[...] = jnp.where(prow == 1, po + h0, po).astype(I32)
        tpos = lax.broadcasted_iota(I32, (TILE, E), 0).astype(F32) * TILE
        eot = jnp.sum((tpos >= po).astype(I32), axis=1) - 1
        eot = jnp.clip(eot, 0, E - 1)
        eot_ref[...] = jnp.broadcast_to(eot[None, :], (8, TILE))


def _gate(x2d, gate_w, gate_b):
    return pl.pallas_call(
        _gate_body,
        grid=(NT,),
        in_specs=[
            pl.BlockSpec((TILE, D), lambda i: (i, 0)),
            pl.BlockSpec((D, E), lambda i: (0, 0)),
            pl.BlockSpec((1, E), lambda i: (0, 0)),
        ],
        out_specs=[
            pl.BlockSpec((TILE, 2), lambda i: (i, 0)),
            pl.BlockSpec((TILE, 2), lambda i: (i, 0)),
            pl.BlockSpec((8, E), lambda i: (0, 0)),
            pl.BlockSpec((8, TILE), lambda i: (0, 0)),
        ],
        out_shape=[
            jax.ShapeDtypeStruct((N, 2), I32),
            jax.ShapeDtypeStruct((N, 2), F32),
            jax.ShapeDtypeStruct((8, E), I32),
            jax.ShapeDtypeStruct((8, TILE), I32),
        ],
        scratch_shapes=[pltpu.VMEM((1, E), F32), pltpu.VMEM((1, E), F32)],
        compiler_params=pltpu.CompilerParams(
            dimension_semantics=("arbitrary",)),
    )(x2d, gate_w, gate_b.reshape(1, E))


DI = D // 2
def _dispatch(xb, code, po2):
    @functools.partial(
        pl.kernel,
        mesh=plsc.VectorSubcoreMesh(core_axis_name="c", subcore_axis_name="s"),
        out_type=[jax.ShapeDtypeStruct((RCAP, D), F32),
                  jax.ShapeDtypeStruct((2, N), I32)],
        scratch_types=[pltpu.VMEM((2, G, D), F32),
                       pltpu.VMEM((G, 2), I32),
                       pltpu.VMEM((4, 2, G), I32),
                       pltpu.VMEM((2, E), I32),
                       pltpu.SemaphoreType.DMA,
                       pltpu.SemaphoreType.DMA,
                       pltpu.SemaphoreType.DMA,
                       pltpu.SemaphoreType.DMA],
        compiler_params=_SC_PARAMS,
    )
    def s1(x_hbm, code_hbm, po_hbm, xs_hbm, dest_hbm,
           rows_v, cd_v, dscr_v, po_v, sl0, sl1, ss0, ss1):
        wid = lax.axis_index("s") * 2 + lax.axis_index("c")
        base0 = wid * G
        base1 = (wid + NSUB) * G
        base2 = (wid + 2 * NSUB) * G
        ld0 = pltpu.make_async_copy(x_hbm.at[pl.ds(base0, G), :],
                                    rows_v.at[0], sl0)
        ld0.start()
        ld1 = pltpu.make_async_copy(x_hbm.at[pl.ds(base1, G), :],
                                    rows_v.at[1], sl1)
        ld1.start()
        pltpu.sync_copy(po_hbm.at[pl.ds(0, 2), :], po_v)

        def compute_dest(base, j):
            pltpu.sync_copy(code_hbm.at[pl.ds(base, G), :], cd_v)
            for k in range(2):
                kf = jnp.full((16,), k, I32)
                for jj in range(G // 16):
                    ridx = lax.iota(I32, 16) + 16 * jj
                    cd = plsc.load_gather(cd_v, [ridx, kf])
                    e = jnp.bitwise_and(cd, E - 1)
                    rk = lax.shift_right_logical(cd, 4)
                    dscr_v[j, k, pl.ds(16 * jj, 16)] = (
                        plsc.load_gather(po_v, [kf, e]) + rk)
            for k in range(2):
                pltpu.sync_copy(dscr_v.at[j, k], dest_hbm.at[k, pl.ds(base, G)])

        compute_dest(base0, 0)
        compute_dest(base1, 1)
        compute_dest(base2, 2)

        def scatter(j, buf, sem):
            hs = [pltpu.make_async_copy(rows_v.at[buf],
                                        xs_hbm.at[dscr_v.at[j, k]], sem)
                  for k in range(2)]
            for h in hs:
                h.start()
            return hs

        ld0.wait()
        sc0 = scatter(0, 0, ss0)
        ld1.wait()
        sc1 = scatter(1, 1, ss1)
        for h in sc0:
            h.wait()
        ld2 = pltpu.make_async_copy(x_hbm.at[pl.ds(base2, G), :],
                                    rows_v.at[0], sl0)
        ld2.start()
        ld2.wait()
        sc2 = scatter(2, 0, ss0)
        for h in sc1:
            h.wait()

        @pl.when(wid < NG - 3 * NSUB)
        def _():
            base3 = (wid + 3 * NSUB) * G
            pltpu.sync_copy(x_hbm.at[pl.ds(base3, G), :], rows_v.at[1])
            compute_dest(base3, 3)
            for k in range(2):
                pltpu.sync_copy(rows_v.at[1], xs_hbm.at[dscr_v.at[3, k]])

        for h in sc2:
            h.wait()

    return s1(xb, code, po2)


def _mlp_body(eot_ref, xs_ref, w1_ref, b1_ref, w2_ref, b2_ref, w3_ref, b3_ref,
              y_ref):
    x = xs_ref[...].astype(jnp.bfloat16)
    h = jnp.dot(x, w1_ref[0], preferred_element_type=F32) + b1_ref[0]
    h = jnp.maximum(h, 0.0).astype(jnp.bfloat16)
    h = jnp.dot(h, w2_ref[0], preferred_element_type=F32) + b2_ref[0]
    h = jnp.maximum(h, 0.0).astype(jnp.bfloat16)
    h = jnp.dot(h, w3_ref[0], preferred_element_type=F32) + b3_ref[0]
    y_ref[...] = jnp.maximum(h, 0.0)


def _moe_mlp(eot, xs, ew1, eb1, ew2, eb2, ew3, eb3):
    grid_spec = pltpu.PrefetchScalarGridSpec(
        num_scalar_prefetch=1,
        grid=(NRT,),
        in_specs=[
            pl.BlockSpec((TILE, D), lambda i, eot: (i, 0)),
            pl.BlockSpec((1, D, H), lambda i, eot: (eot[0, i], 0, 0)),
            pl.BlockSpec((1, 1, H), lambda i, eot: (eot[0, i], 0, 0)),
            pl.BlockSpec((1, H, H), lambda i, eot: (eot[0, i], 0, 0)),
            pl.BlockSpec((1, 1, H), lambda i, eot: (eot[0, i], 0, 0)),
            pl.BlockSpec((1, H, H), lambda i, eot: (eot[0, i], 0, 0)),
            pl.BlockSpec((1, 1, H), lambda i, eot: (eot[0, i], 0, 0)),
        ],
        out_specs=pl.BlockSpec((TILE, H), lambda i, eot: (i, 0)),
    )
    return pl.pallas_call(
        _mlp_body,
        grid_spec=grid_spec,
        out_shape=jax.ShapeDtypeStruct((RCAP, H), F32),
        compiler_params=pltpu.CompilerParams(
            dimension_semantics=("arbitrary",)),
    )(eot, xs, ew1.astype(jnp.bfloat16), eb1.reshape(E, 1, H),
      ew2.astype(jnp.bfloat16), eb2.reshape(E, 1, H),
      ew3.astype(jnp.bfloat16), eb3.reshape(E, 1, H))


def _gather_pairs(dest, y):
    @functools.partial(
        pl.kernel,
        mesh=plsc.VectorSubcoreMesh(core_axis_name="c", subcore_axis_name="s"),
        out_type=jax.ShapeDtypeStruct((2, N, H), F32),
        scratch_types=[pltpu.VMEM((8, G, H), F32),
                       pltpu.VMEM((4, 2, G), I32),
                       pltpu.SemaphoreType.DMA,
                       pltpu.SemaphoreType.DMA],
        compiler_params=_SC_PARAMS,
    )
    def s2(dest_hbm, y_hbm, z_hbm, z_v, d_v, sg, sw):
        wid = lax.axis_index("s") * 2 + lax.axis_index("c")
        gathers = []
        for j in range(3):
            base = (wid + NSUB * j) * G
            for k in range(2):
                pltpu.sync_copy(dest_hbm.at[k, pl.ds(base, G)], d_v.at[j, k])
                gathers.append(pltpu.make_async_copy(
                    y_hbm.at[d_v.at[j, k]], z_v.at[2 * j + k], sg))
                gathers[-1].start()
        for h in gathers:
            h.wait()
        writes = []
        for j in range(3):
            base = (wid + NSUB * j) * G
            for k in range(2):
                writes.append(pltpu.make_async_copy(
                    z_v.at[2 * j + k], z_hbm.at[k, pl.ds(base, G), :], sw))
                writes[-1].start()

        @pl.when(wid < NG - 3 * NSUB)
        def _():
            base = (wid + 3 * NSUB) * G
            for k in range(2):
                pltpu.sync_copy(dest_hbm.at[k, pl.ds(base, G)], d_v.at[3, k])
                pltpu.sync_copy(y_hbm.at[d_v.at[3, k]], z_v.at[6 + k])
                pltpu.sync_copy(z_v.at[6 + k], z_hbm.at[k, pl.ds(base, G), :])

        for h in writes:
            h.wait()

    return s2(dest, y)


def _combine_body(z_ref, w_ref, w2_ref, b2_ref, w3_ref, b3_ref, y_ref):
    w = w_ref[...]
    a = w[:, 0:1] * z_ref[0] + w[:, 1:2] * z_ref[1]
    a = jnp.maximum(a, 0.0).astype(jnp.bfloat16)
    y = jnp.dot(a, w2_ref[...], preferred_element_type=F32) + b2_ref[...]
    y = jnp.maximum(y, 0.0).astype(jnp.bfloat16)
    y = jnp.dot(y, w3_ref[...], preferred_element_type=F32) + b3_ref[...]
    y_ref[...] = jnp.maximum(y, 0.0).astype(jnp.bfloat16)


def _combine(z, wco, fc2_w, fc2_b, fc3_w, fc3_b):
    return pl.pallas_call(
        _combine_body,
        grid=(NT,),
        in_specs=[
            pl.BlockSpec((2, TILE, H), lambda i: (0, i, 0)),
            pl.BlockSpec((TILE, 2), lambda i: (i, 0)),
            pl.BlockSpec((H, H), lambda i: (0, 0)),
            pl.BlockSpec((1, H), lambda i: (0, 0)),
            pl.BlockSpec((H, H), lambda i: (0, 0)),
            pl.BlockSpec((1, H), lambda i: (0, 0)),
        ],
        out_specs=pl.BlockSpec((TILE, H), lambda i: (i, 0)),
        out_shape=jax.ShapeDtypeStruct((N, H), jnp.bfloat16),
        compiler_params=pltpu.CompilerParams(
            dimension_semantics=("arbitrary",)),
    )(z, wco, fc2_w.astype(jnp.bfloat16), fc2_b.reshape(1, H),
      fc3_w.astype(jnp.bfloat16), fc3_b.reshape(1, H))


TK = 28
NKC = T // TK


def _tail_body(y_ref, w4_ref, b4_ref, w5_ref, b5_ref, w6_ref, b6_ref,
               o_ref, acc_ref):
    i = pl.program_id(0)

    @pl.when(i == 0)
    def _():
        acc_ref[...] = jnp.zeros_like(acc_ref)

    acc = acc_ref[...]
    for tk in range(TK):
        acc += jnp.dot(y_ref[tk], w4_ref[tk].astype(jnp.bfloat16),
                       preferred_element_type=F32)
    acc_ref[...] = acc

    @pl.when(i == NKC - 1)
    def _():
        z = jnp.maximum(acc_ref[...] + b4_ref[...], 0.0).astype(jnp.bfloat16)
        z = jnp.dot(z, w5_ref[...].astype(jnp.bfloat16),
                    preferred_element_type=F32) + b5_ref[...]
        z = jnp.maximum(z, 0.0).astype(jnp.bfloat16)
        o_ref[...] = jnp.dot(z, w6_ref[...].astype(jnp.bfloat16),
                             preferred_element_type=F32) + b6_ref[...]


def _tail(y1t, fc4_w, fc4_b, fc5_w, fc5_b, fc6_w, fc6_b):
    return pl.pallas_call(
        _tail_body,
        grid=(NKC,),
        in_specs=[
            pl.BlockSpec((TK, B, H), lambda i: (i, 0, 0)),
            pl.BlockSpec((TK, H, H), lambda i: (i, 0, 0)),
            pl.BlockSpec((1, H), lambda i: (0, 0)),
            pl.BlockSpec((H, H), lambda i: (0, 0)),
            pl.BlockSpec((1, H), lambda i: (0, 0)),
            pl.BlockSpec((H, OUT), lambda i: (0, 0)),
            pl.BlockSpec((1, OUT), lambda i: (0, 0)),
        ],
        out_specs=pl.BlockSpec((B, OUT), lambda i: (0, 0)),
        out_shape=jax.ShapeDtypeStruct((B, OUT), F32),
        scratch_shapes=[pltpu.VMEM((B, H), F32)],
        compiler_params=pltpu.CompilerParams(
            dimension_semantics=("arbitrary",)),
    )(y1t, fc4_w.reshape(T, H, H), fc4_b.reshape(1, H),
      fc5_w, fc5_b.reshape(1, H), fc6_w, fc6_b.reshape(1, OUT))


def kernel(x, gate_w, gate_b, ew1, eb1, ew2, eb2, ew3, eb3,
           fc2_w, fc2_b, fc3_w, fc3_b, fc4_w, fc4_b,
           fc5_w, fc5_b, fc6_w, fc6_b):
    x2d = jnp.transpose(x, (1, 0, 2)).reshape(N, D)
    code, wco, po2, eot_pad = _gate(x2d, gate_w, gate_b)
    xs, dest = _dispatch(x2d, code, po2)
    y = _moe_mlp(eot_pad, xs, ew1, eb1, ew2, eb2, ew3, eb3)
    z = _gather_pairs(dest, y)
    y1 = _combine(z, wco, fc2_w, fc2_b, fc3_w, fc3_b)
    return _tail(y1.reshape(T, B, H), fc4_w, fc4_b, fc5_w, fc5_b,
                 fc6_w, fc6_b)


# speedup vs baseline: 1.2981x; 1.1317x over previous
import dataclasses
import functools

import jax
import jax.numpy as jnp
from jax import lax
from jax.experimental import pallas as pl
from jax.experimental.pallas import tpu as pltpu
from jax.experimental.pallas import tpu_sc as plsc

B, T, D, E, K, H, OUT = 32, 196, 768, 16, 2, 128, 18
N = B * T
TILE = 128
NT = N // TILE
RCAP = (N * K // TILE + E) * TILE
NRT = RCAP // TILE
G = 64
NG = N // G
NSUB = 32
F32 = jnp.float32
I32 = jnp.int32

_SC_PARAMS = pltpu.CompilerParams()
if "needs_layout_passes" in pltpu.CompilerParams.__dataclass_fields__:
    _SC_PARAMS = dataclasses.replace(_SC_PARAMS, needs_layout_passes=False)


def _gate_body(x_ref, gw_ref, gb_ref, code_ref, w_ref,
               po_ref, eot_ref, c0_ref, c1_ref):
    i = pl.program_id(0)

    @pl.when(i == 0)
    def _():
        c0_ref[...] = jnp.zeros_like(c0_ref)
        c1_ref[...] = jnp.zeros_like(c1_ref)

    x = x_ref[...]
    logits = jnp.dot(x, gw_ref[...], preferred_element_type=F32)
    logits = logits + gb_ref[...]
    m = jnp.max(logits, axis=1, keepdims=True)
    ex = jnp.exp(logits - m)
    probs = ex / jnp.sum(ex, axis=1, keepdims=True)

    idx = lax.broadcasted_iota(I32, (TILE, E), 1)
    m0 = jnp.max(probs, axis=1, keepdims=True)
    e0 = jnp.min(jnp.where(probs >= m0, idx, E + 1), axis=1)
    oh0 = (idx == e0[:, None]).astype(F32)
    probs2 = jnp.where(oh0 > 0, -1.0, probs)
    m1 = jnp.max(probs2, axis=1, keepdims=True)
    e1 = jnp.min(jnp.where(probs2 >= m1, idx, E + 1), axis=1)
    oh1 = (idx == e1[:, None]).astype(F32)

    r = lax.broadcasted_iota(I32, (TILE, TILE), 0)
    c = lax.broadcasted_iota(I32, (TILE, TILE), 1)
    lt = (r > c).astype(F32)
    ex0 = jnp.dot(lt, oh0, preferred_element_type=F32)
    ex1 = jnp.dot(lt, oh1, preferred_element_type=F32)
    rank0 = jnp.sum(ex0 * oh0, axis=1) + jnp.sum(oh0 * c0_ref[...], axis=1)
    rank1 = jnp.sum(ex1 * oh1, axis=1) + jnp.sum(oh1 * c1_ref[...], axis=1)
    c0_ref[...] = c0_ref[...] + jnp.sum(oh0, axis=0, keepdims=True)
    c1_ref[...] = c1_ref[...] + jnp.sum(oh1, axis=0, keepdims=True)

    col = lax.broadcasted_iota(I32, (TILE, 2), 1)
    code0 = (rank0.astype(I32) << 4) | e0
    code1 = (rank1.astype(I32) << 4) | e1
    code_ref[...] = jnp.where(col == 0, code0[:, None], code1[:, None])
    w_ref[...] = jnp.where(col == 0, probs[:, 0:1], probs[:, 1:2])

    @pl.when(i == NT - 1)
    def _():
        h0 = c0_ref[...]
        ht = c0_ref[...] + c1_ref[...]
        padded = jnp.ceil(ht / TILE) * TILE
        rr = lax.broadcasted_iota(I32, (E, E), 0)
        cc = lax.broadcasted_iota(I32, (E, E), 1)
        su = (rr < cc).astype(F32)
        po = jnp.dot(padded, su, preferred_element_type=F32)
        prow = lax.broadcasted_iota(I32, (8, E), 0)
        po_ref[...] = jnp.where(prow == 1, po + h0, po).astype(I32)
        tpos = lax.broadcasted_iota(I32, (TILE, E), 0).astype(F32) * TILE
        eot = jnp.sum((tpos >= po).astype(I32), axis=1) - 1
        eot = jnp.clip(eot, 0, E - 1)
        eot_ref[...] = jnp.broadcast_to(eot[None, :], (8, TILE))


def _gate(x2d, gate_w, gate_b):
    return pl.pallas_call(
        _gate_body,
        grid=(NT,),
        in_specs=[
            pl.BlockSpec((TILE, D), lambda i: (i, 0)),
            pl.BlockSpec((D, E), lambda i: (0, 0)),
            pl.BlockSpec((1, E), lambda i: (0, 0)),
        ],
        out_specs=[
            pl.BlockSpec((TILE, 2), lambda i: (i, 0)),
            pl.BlockSpec((TILE, 2), lambda i: (i, 0)),
            pl.BlockSpec((8, E), lambda i: (0, 0)),
            pl.BlockSpec((8, TILE), lambda i: (0, 0)),
        ],
        out_shape=[
            jax.ShapeDtypeStruct((N, 2), I32),
            jax.ShapeDtypeStruct((N, 2), F32),
            jax.ShapeDtypeStruct((8, E), I32),
            jax.ShapeDtypeStruct((8, TILE), I32),
        ],
        scratch_shapes=[pltpu.VMEM((1, E), F32), pltpu.VMEM((1, E), F32)],
        compiler_params=pltpu.CompilerParams(
            dimension_semantics=("arbitrary",)),
    )(x2d, gate_w, gate_b.reshape(1, E))


DI = D // 2
def _dispatch(xb, code, po2):
    @functools.partial(
        pl.kernel,
        mesh=plsc.VectorSubcoreMesh(core_axis_name="c", subcore_axis_name="s"),
        out_type=[jax.ShapeDtypeStruct((RCAP, D), F32),
                  jax.ShapeDtypeStruct((2, N), I32)],
        scratch_types=[pltpu.VMEM((2, G, D), F32),
                       pltpu.VMEM((G, 2), I32),
                       pltpu.VMEM((4, 2, G), I32),
                       pltpu.VMEM((2, E), I32),
                       pltpu.SemaphoreType.DMA,
                       pltpu.SemaphoreType.DMA,
                       pltpu.SemaphoreType.DMA,
                       pltpu.SemaphoreType.DMA],
        compiler_params=_SC_PARAMS,
    )
    def s1(x_hbm, code_hbm, po_hbm, xs_hbm, dest_hbm,
           rows_v, cd_v, dscr_v, po_v, sl0, sl1, ss0, ss1):
        wid = lax.axis_index("s") * 2 + lax.axis_index("c")
        base0 = wid * G
        base1 = (wid + NSUB) * G
        base2 = (wid + 2 * NSUB) * G
        ld0 = pltpu.make_async_copy(x_hbm.at[pl.ds(base0, G), :],
                                    rows_v.at[0], sl0)
        ld0.start()
        ld1 = pltpu.make_async_copy(x_hbm.at[pl.ds(base1, G), :],
                                    rows_v.at[1], sl1)
        ld1.start()
        pltpu.sync_copy(po_hbm.at[pl.ds(0, 2), :], po_v)

        def compute_dest(base, j):
            pltpu.sync_copy(code_hbm.at[pl.ds(base, G), :], cd_v)
            for k in range(2):
                kf = jnp.full((16,), k, I32)
                for jj in range(G // 16):
                    ridx = lax.iota(I32, 16) + 16 * jj
                    cd = plsc.load_gather(cd_v, [ridx, kf])
                    e = jnp.bitwise_and(cd, E - 1)
                    rk = lax.shift_right_logical(cd, 4)
                    dscr_v[j, k, pl.ds(16 * jj, 16)] = (
                        plsc.load_gather(po_v, [kf, e]) + rk)
            for k in range(2):
                pltpu.sync_copy(dscr_v.at[j, k], dest_hbm.at[k, pl.ds(base, G)])

        compute_dest(base0, 0)
        compute_dest(base1, 1)
        compute_dest(base2, 2)

        def scatter(j, buf, sem):
            hs = [pltpu.make_async_copy(rows_v.at[buf],
                                        xs_hbm.at[dscr_v.at[j, k]], sem)
                  for k in range(2)]
            for h in hs:
                h.start()
            return hs

        ld0.wait()
        sc0 = scatter(0, 0, ss0)
        ld1.wait()
        sc1 = scatter(1, 1, ss1)
        for h in sc0:
            h.wait()
        ld2 = pltpu.make_async_copy(x_hbm.at[pl.ds(base2, G), :],
                                    rows_v.at[0], sl0)
        ld2.start()
        ld2.wait()
        sc2 = scatter(2, 0, ss0)
        for h in sc1:
            h.wait()

        @pl.when(wid < NG - 3 * NSUB)
        def _():
            base3 = (wid + 3 * NSUB) * G
            pltpu.sync_copy(x_hbm.at[pl.ds(base3, G), :], rows_v.at[1])
            compute_dest(base3, 3)
            for k in range(2):
                pltpu.sync_copy(rows_v.at[1], xs_hbm.at[dscr_v.at[3, k]])

        for h in sc2:
            h.wait()

    return s1(xb, code, po2)


def _mlp_body(eot_ref, xs_ref, w1_ref, b1_ref, w2_ref, b2_ref, w3_ref, b3_ref,
              y_ref):
    x = xs_ref[...].astype(jnp.bfloat16)
    h = jnp.dot(x, w1_ref[0], preferred_element_type=F32) + b1_ref[0]
    h = jnp.maximum(h, 0.0).astype(jnp.bfloat16)
    h = jnp.dot(h, w2_ref[0], preferred_element_type=F32) + b2_ref[0]
    h = jnp.maximum(h, 0.0).astype(jnp.bfloat16)
    h = jnp.dot(h, w3_ref[0], preferred_element_type=F32) + b3_ref[0]
    y_ref[...] = jnp.maximum(h, 0.0)


def _moe_mlp(eot, xs, ew1, eb1, ew2, eb2, ew3, eb3):
    grid_spec = pltpu.PrefetchScalarGridSpec(
        num_scalar_prefetch=1,
        grid=(NRT,),
        in_specs=[
            pl.BlockSpec((TILE, D), lambda i, eot: (i, 0)),
            pl.BlockSpec((1, D, H), lambda i, eot: (eot[0, i], 0, 0)),
            pl.BlockSpec((1, 1, H), lambda i, eot: (eot[0, i], 0, 0)),
            pl.BlockSpec((1, H, H), lambda i, eot: (eot[0, i], 0, 0)),
            pl.BlockSpec((1, 1, H), lambda i, eot: (eot[0, i], 0, 0)),
            pl.BlockSpec((1, H, H), lambda i, eot: (eot[0, i], 0, 0)),
            pl.BlockSpec((1, 1, H), lambda i, eot: (eot[0, i], 0, 0)),
        ],
        out_specs=pl.BlockSpec((TILE, H), lambda i, eot: (i, 0)),
    )
    return pl.pallas_call(
        _mlp_body,
        grid_spec=grid_spec,
        out_shape=jax.ShapeDtypeStruct((RCAP, H), F32),
        compiler_params=pltpu.CompilerParams(
            dimension_semantics=("arbitrary",)),
    )(eot, xs, ew1.astype(jnp.bfloat16), eb1.reshape(E, 1, H),
      ew2.astype(jnp.bfloat16), eb2.reshape(E, 1, H),
      ew3.astype(jnp.bfloat16), eb3.reshape(E, 1, H))


def _gather_pairs(dest, y):
    @functools.partial(
        pl.kernel,
        mesh=plsc.VectorSubcoreMesh(core_axis_name="c", subcore_axis_name="s"),
        out_type=jax.ShapeDtypeStruct((2, N, H), F32),
        scratch_types=[pltpu.VMEM((8, G, H), F32),
                       pltpu.VMEM((4, 2, G), I32),
                       pltpu.SemaphoreType.DMA,
                       pltpu.SemaphoreType.DMA],
        compiler_params=_SC_PARAMS,
    )
    def s2(dest_hbm, y_hbm, z_hbm, z_v, d_v, sg, sw):
        wid = lax.axis_index("s") * 2 + lax.axis_index("c")
        gathers = []
        for j in range(3):
            base = (wid + NSUB * j) * G
            for k in range(2):
                pltpu.sync_copy(dest_hbm.at[k, pl.ds(base, G)], d_v.at[j, k])
                gathers.append(pltpu.make_async_copy(
                    y_hbm.at[d_v.at[j, k]], z_v.at[2 * j + k], sg))
                gathers[-1].start()
        for h in gathers:
            h.wait()
        writes = []
        for j in range(3):
            base = (wid + NSUB * j) * G
            for k in range(2):
                writes.append(pltpu.make_async_copy(
                    z_v.at[2 * j + k], z_hbm.at[k, pl.ds(base, G), :], sw))
                writes[-1].start()

        @pl.when(wid < NG - 3 * NSUB)
        def _():
            base = (wid + 3 * NSUB) * G
            for k in range(2):
                pltpu.sync_copy(dest_hbm.at[k, pl.ds(base, G)], d_v.at[3, k])
                pltpu.sync_copy(y_hbm.at[d_v.at[3, k]], z_v.at[6 + k])
                pltpu.sync_copy(z_v.at[6 + k], z_hbm.at[k, pl.ds(base, G), :])

        for h in writes:
            h.wait()

    return s2(dest, y)


TK = 28
NKC = T // TK
TB = TK * B


def _cd_body(z_ref, w_ref, w2_ref, b2_ref, w3_ref, b3_ref,
             w4_ref, b4_ref, w5_ref, b5_ref, w6_ref, b6_ref,
             o_ref, acc_ref):
    i = pl.program_id(0)

    @pl.when(i == 0)
    def _():
        acc_ref[...] = jnp.zeros_like(acc_ref)

    w = w_ref[...]
    a = w[:, 0:1] * z_ref[0] + w[:, 1:2] * z_ref[1]
    a = jnp.maximum(a, 0.0).astype(jnp.bfloat16)
    y = jnp.dot(a, w2_ref[...], preferred_element_type=F32) + b2_ref[...]
    y = jnp.maximum(y, 0.0).astype(jnp.bfloat16)
    y = jnp.dot(y, w3_ref[...], preferred_element_type=F32) + b3_ref[...]
    y = jnp.maximum(y, 0.0).astype(jnp.bfloat16)
    y = y.reshape(TK, B, H)
    acc = acc_ref[...]
    for tk in range(TK):
        acc += jnp.dot(y[tk], w4_ref[tk].astype(jnp.bfloat16),
                       preferred_element_type=F32)
    acc_ref[...] = acc

    @pl.when(i == NKC - 1)
    def _():
        zz = jnp.maximum(acc_ref[...] + b4_ref[...], 0.0).astype(jnp.bfloat16)
        zz = jnp.dot(zz, w5_ref[...].astype(jnp.bfloat16),
                     preferred_element_type=F32) + b5_ref[...]
        zz = jnp.maximum(zz, 0.0).astype(jnp.bfloat16)
        o_ref[...] = jnp.dot(zz, w6_ref[...].astype(jnp.bfloat16),
                             preferred_element_type=F32) + b6_ref[...]


def _cd(z, wco, fc2_w, fc2_b, fc3_w, fc3_b, fc4_w, fc4_b,
        fc5_w, fc5_b, fc6_w, fc6_b):
    return pl.pallas_call(
        _cd_body,
        grid=(NKC,),
        in_specs=[
            pl.BlockSpec((2, TB, H), lambda i: (0, i, 0)),
            pl.BlockSpec((TB, 2), lambda i: (i, 0)),
            pl.BlockSpec((H, H), lambda i: (0, 0)),
            pl.BlockSpec((1, H), lambda i: (0, 0)),
            pl.BlockSpec((H, H), lambda i: (0, 0)),
            pl.BlockSpec((1, H), lambda i: (0, 0)),
            pl.BlockSpec((TK, H, H), lambda i: (i, 0, 0)),
            pl.BlockSpec((1, H), lambda i: (0, 0)),
            pl.BlockSpec((H, H), lambda i: (0, 0)),
            pl.BlockSpec((1, H), lambda i: (0, 0)),
            pl.BlockSpec((H, OUT), lambda i: (0, 0)),
            pl.BlockSpec((1, OUT), lambda i: (0, 0)),
        ],
        out_specs=pl.BlockSpec((B, OUT), lambda i: (0, 0)),
        out_shape=jax.ShapeDtypeStruct((B, OUT), F32),
        scratch_shapes=[pltpu.VMEM((B, H), F32)],
        compiler_params=pltpu.CompilerParams(
            dimension_semantics=("arbitrary",)),
    )(z, wco, fc2_w.astype(jnp.bfloat16), fc2_b.reshape(1, H),
      fc3_w.astype(jnp.bfloat16), fc3_b.reshape(1, H),
      fc4_w.reshape(T, H, H), fc4_b.reshape(1, H),
      fc5_w, fc5_b.reshape(1, H), fc6_w, fc6_b.reshape(1, OUT))


def kernel(x, gate_w, gate_b, ew1, eb1, ew2, eb2, ew3, eb3,
           fc2_w, fc2_b, fc3_w, fc3_b, fc4_w, fc4_b,
           fc5_w, fc5_b, fc6_w, fc6_b):
    x2d = jnp.transpose(x, (1, 0, 2)).reshape(N, D)
    code, wco, po2, eot_pad = _gate(x2d, gate_w, gate_b)
    xs, dest = _dispatch(x2d, code, po2)
    y = _moe_mlp(eot_pad, xs, ew1, eb1, ew2, eb2, ew3, eb3)
    z = _gather_pairs(dest, y)
    return _cd(z, wco, fc2_w, fc2_b, fc3_w, fc3_b, fc4_w, fc4_b,
               fc5_w, fc5_b, fc6_w, fc6_b)


# speedup vs baseline: 1.4142x; 1.0894x over previous
import dataclasses
import functools

import jax
import jax.numpy as jnp
from jax import lax
from jax.experimental import pallas as pl
from jax.experimental.pallas import tpu as pltpu
from jax.experimental.pallas import tpu_sc as plsc

B, T, D, E, K, H, OUT = 32, 196, 768, 16, 2, 128, 18
N = B * T
TILE = 128
NT = N // TILE
GT = 224
NGT = N // GT
RCAP = (N * K // TILE + E) * TILE
NRT = RCAP // TILE
G = 64
NG = N // G
NSUB = 32
F32 = jnp.float32
I32 = jnp.int32

_SC_PARAMS = pltpu.CompilerParams()
if "needs_layout_passes" in pltpu.CompilerParams.__dataclass_fields__:
    _SC_PARAMS = dataclasses.replace(_SC_PARAMS, needs_layout_passes=False)


def _gate_body(x_ref, gw_ref, gb_ref, code_ref, w_ref,
               po_ref, eot_ref, c0_ref, c1_ref):
    i = pl.program_id(0)

    @pl.when(i == 0)
    def _():
        c0_ref[...] = jnp.zeros_like(c0_ref)
        c1_ref[...] = jnp.zeros_like(c1_ref)

    x = x_ref[...]
    logits = jnp.dot(x, gw_ref[...], preferred_element_type=F32)
    logits = logits + gb_ref[...]
    m = jnp.max(logits, axis=1, keepdims=True)
    ex = jnp.exp(logits - m)
    probs = ex / jnp.sum(ex, axis=1, keepdims=True)

    idx = lax.broadcasted_iota(I32, (GT, E), 1)
    m0 = jnp.max(probs, axis=1, keepdims=True)
    e0 = jnp.min(jnp.where(probs >= m0, idx, E + 1), axis=1)
    oh0 = (idx == e0[:, None]).astype(F32)
    probs2 = jnp.where(oh0 > 0, -1.0, probs)
    m1 = jnp.max(probs2, axis=1, keepdims=True)
    e1 = jnp.min(jnp.where(probs2 >= m1, idx, E + 1), axis=1)
    oh1 = (idx == e1[:, None]).astype(F32)

    r = lax.broadcasted_iota(I32, (GT, GT), 0)
    c = lax.broadcasted_iota(I32, (GT, GT), 1)
    lt = (r > c).astype(F32)
    ex0 = jnp.dot(lt, oh0, preferred_element_type=F32)
    ex1 = jnp.dot(lt, oh1, preferred_element_type=F32)
    rank0 = jnp.sum(ex0 * oh0, axis=1) + jnp.sum(oh0 * c0_ref[...], axis=1)
    rank1 = jnp.sum(ex1 * oh1, axis=1) + jnp.sum(oh1 * c1_ref[...], axis=1)
    c0_ref[...] = c0_ref[...] + jnp.sum(oh0, axis=0, keepdims=True)
    c1_ref[...] = c1_ref[...] + jnp.sum(oh1, axis=0, keepdims=True)

    col = lax.broadcasted_iota(I32, (GT, 2), 1)
    code0 = (rank0.astype(I32) << 4) | e0
    code1 = (rank1.astype(I32) << 4) | e1
    code_ref[...] = jnp.where(col == 0, code0[:, None], code1[:, None])
    w_ref[...] = jnp.where(col == 0, probs[:, 0:1], probs[:, 1:2])

    @pl.when(i == NGT - 1)
    def _():
        h0 = c0_ref[...]
        ht = c0_ref[...] + c1_ref[...]
        padded = jnp.ceil(ht / TILE) * TILE
        rr = lax.broadcasted_iota(I32, (E, E), 0)
        cc = lax.broadcasted_iota(I32, (E, E), 1)
        su = (rr < cc).astype(F32)
        po = jnp.dot(padded, su, preferred_element_type=F32)
        prow = lax.broadcasted_iota(I32, (8, E), 0)
        po_ref[...] = jnp.where(prow == 1, po + h0, po).astype(I32)
        tpos = lax.broadcasted_iota(I32, (TILE, E), 0).astype(F32) * TILE
        eot = jnp.sum((tpos >= po).astype(I32), axis=1) - 1
        eot = jnp.clip(eot, 0, E - 1)
        eot_ref[...] = jnp.broadcast_to(eot[None, :], (8, TILE))


def _gate(x2d, gate_w, gate_b):
    return pl.pallas_call(
        _gate_body,
        grid=(NGT,),
        in_specs=[
            pl.BlockSpec((GT, D), lambda i: (i, 0)),
            pl.BlockSpec((D, E), lambda i: (0, 0)),
            pl.BlockSpec((1, E), lambda i: (0, 0)),
        ],
        out_specs=[
            pl.BlockSpec((GT, 2), lambda i: (i, 0)),
            pl.BlockSpec((GT, 2), lambda i: (i, 0)),
            pl.BlockSpec((8, E), lambda i: (0, 0)),
            pl.BlockSpec((8, TILE), lambda i: (0, 0)),
        ],
        out_shape=[
            jax.ShapeDtypeStruct((N, 2), I32),
            jax.ShapeDtypeStruct((N, 2), F32),
            jax.ShapeDtypeStruct((8, E), I32),
            jax.ShapeDtypeStruct((8, TILE), I32),
        ],
        scratch_shapes=[pltpu.VMEM((1, E), F32), pltpu.VMEM((1, E), F32)],
        compiler_params=pltpu.CompilerParams(
            dimension_semantics=("arbitrary",)),
    )(x2d, gate_w, gate_b.reshape(1, E))


DI = D // 2
def _dispatch(xb, code, po2):
    @functools.partial(
        pl.kernel,
        mesh=plsc.VectorSubcoreMesh(core_axis_name="c", subcore_axis_name="s"),
        out_type=[jax.ShapeDtypeStruct((RCAP, D), F32),
                  jax.ShapeDtypeStruct((2, N), I32)],
        scratch_types=[pltpu.VMEM((2, G, D), F32),
                       pltpu.VMEM((G, 2), I32),
                       pltpu.VMEM((4, 2, G), I32),
                       pltpu.VMEM((2, E), I32),
                       pltpu.SemaphoreType.DMA,
                       pltpu.SemaphoreType.DMA,
                       pltpu.SemaphoreType.DMA,
                       pltpu.SemaphoreType.DMA],
        compiler_params=_SC_PARAMS,
    )
    def s1(x_hbm, code_hbm, po_hbm, xs_hbm, dest_hbm,
           rows_v, cd_v, dscr_v, po_v, sl0, sl1, ss0, ss1):
        wid = lax.axis_index("s") * 2 + lax.axis_index("c")
        base0 = wid * G
        base1 = (wid + NSUB) * G
        base2 = (wid + 2 * NSUB) * G
        ld0 = pltpu.make_async_copy(x_hbm.at[pl.ds(base0, G), :],
                                    rows_v.at[0], sl0)
        ld0.start()
        ld1 = pltpu.make_async_copy(x_hbm.at[pl.ds(base1, G), :],
                                    rows_v.at[1], sl1)
        ld1.start()
        pltpu.sync_copy(po_hbm.at[pl.ds(0, 2), :], po_v)

        def compute_dest(base, j):
            pltpu.sync_copy(code_hbm.at[pl.ds(base, G), :], cd_v)
            for k in range(2):
                kf = jnp.full((16,), k, I32)
                for jj in range(G // 16):
                    ridx = lax.iota(I32, 16) + 16 * jj
                    cd = plsc.load_gather(cd_v, [ridx, kf])
                    e = jnp.bitwise_and(cd, E - 1)
                    rk = lax.shift_right_logical(cd, 4)
                    dscr_v[j, k, pl.ds(16 * jj, 16)] = (
                        plsc.load_gather(po_v, [kf, e]) + rk)
            for k in range(2):
                pltpu.sync_copy(dscr_v.at[j, k], dest_hbm.at[k, pl.ds(base, G)])

        compute_dest(base0, 0)
        compute_dest(base1, 1)
        compute_dest(base2, 2)

        def scatter(j, buf, sem):
            hs = [pltpu.make_async_copy(rows_v.at[buf],
                                        xs_hbm.at[dscr_v.at[j, k]], sem)
                  for k in range(2)]
            for h in hs:
                h.start()
            return hs

        ld0.wait()
        sc0 = scatter(0, 0, ss0)
        ld1.wait()
        sc1 = scatter(1, 1, ss1)
        for h in sc0:
            h.wait()
        ld2 = pltpu.make_async_copy(x_hbm.at[pl.ds(base2, G), :],
                                    rows_v.at[0], sl0)
        ld2.start()
        ld2.wait()
        sc2 = scatter(2, 0, ss0)
        for h in sc1:
            h.wait()

        @pl.when(wid < NG - 3 * NSUB)
        def _():
            base3 = (wid + 3 * NSUB) * G
            pltpu.sync_copy(x_hbm.at[pl.ds(base3, G), :], rows_v.at[1])
            compute_dest(base3, 3)
            for k in range(2):
                pltpu.sync_copy(rows_v.at[1], xs_hbm.at[dscr_v.at[3, k]])

        for h in sc2:
            h.wait()

    return s1(xb, code, po2)


def _mlp_body(eot_ref, xs_ref, w1_ref, b1_ref, w2_ref, b2_ref, w3_ref, b3_ref,
              y_ref):
    x = xs_ref[...].astype(jnp.bfloat16)
    h = jnp.dot(x, w1_ref[0], preferred_element_type=F32) + b1_ref[0]
    h = jnp.maximum(h, 0.0).astype(jnp.bfloat16)
    h = jnp.dot(h, w2_ref[0], preferred_element_type=F32) + b2_ref[0]
    h = jnp.maximum(h, 0.0).astype(jnp.bfloat16)
    h = jnp.dot(h, w3_ref[0], preferred_element_type=F32) + b3_ref[0]
    y_ref[...] = jnp.maximum(h, 0.0)


def _moe_mlp(eot, xs, ew1, eb1, ew2, eb2, ew3, eb3):
    grid_spec = pltpu.PrefetchScalarGridSpec(
        num_scalar_prefetch=1,
        grid=(NRT,),
        in_specs=[
            pl.BlockSpec((TILE, D), lambda i, eot: (i, 0)),
            pl.BlockSpec((1, D, H), lambda i, eot: (eot[0, i], 0, 0)),
            pl.BlockSpec((1, 1, H), lambda i, eot: (eot[0, i], 0, 0)),
            pl.BlockSpec((1, H, H), lambda i, eot: (eot[0, i], 0, 0)),
            pl.BlockSpec((1, 1, H), lambda i, eot: (eot[0, i], 0, 0)),
            pl.BlockSpec((1, H, H), lambda i, eot: (eot[0, i], 0, 0)),
            pl.BlockSpec((1, 1, H), lambda i, eot: (eot[0, i], 0, 0)),
        ],
        out_specs=pl.BlockSpec((TILE, H), lambda i, eot: (i, 0)),
    )
    return pl.pallas_call(
        _mlp_body,
        grid_spec=grid_spec,
        out_shape=jax.ShapeDtypeStruct((RCAP, H), F32),
        compiler_params=pltpu.CompilerParams(
            dimension_semantics=("arbitrary",)),
    )(eot, xs, ew1.astype(jnp.bfloat16), eb1.reshape(E, 1, H),
      ew2.astype(jnp.bfloat16), eb2.reshape(E, 1, H),
      ew3.astype(jnp.bfloat16), eb3.reshape(E, 1, H))


def _gather_pairs(dest, y):
    @functools.partial(
        pl.kernel,
        mesh=plsc.VectorSubcoreMesh(core_axis_name="c", subcore_axis_name="s"),
        out_type=jax.ShapeDtypeStruct((2, N, H), F32),
        scratch_types=[pltpu.VMEM((8, G, H), F32),
                       pltpu.VMEM((4, 2, G), I32),
                       pltpu.SemaphoreType.DMA,
                       pltpu.SemaphoreType.DMA],
        compiler_params=_SC_PARAMS,
    )
    def s2(dest_hbm, y_hbm, z_hbm, z_v, d_v, sg, sw):
        wid = lax.axis_index("s") * 2 + lax.axis_index("c")
        gathers = []
        for j in range(3):
            base = (wid + NSUB * j) * G
            for k in range(2):
                pltpu.sync_copy(dest_hbm.at[k, pl.ds(base, G)], d_v.at[j, k])
                gathers.append(pltpu.make_async_copy(
                    y_hbm.at[d_v.at[j, k]], z_v.at[2 * j + k], sg))
                gathers[-1].start()
        for h in gathers:
            h.wait()
        writes = []
        for j in range(3):
            base = (wid + NSUB * j) * G
            for k in range(2):
                writes.append(pltpu.make_async_copy(
                    z_v.at[2 * j + k], z_hbm.at[k, pl.ds(base, G), :], sw))
                writes[-1].start()

        @pl.when(wid < NG - 3 * NSUB)
        def _():
            base = (wid + 3 * NSUB) * G
            for k in range(2):
                pltpu.sync_copy(dest_hbm.at[k, pl.ds(base, G)], d_v.at[3, k])
                pltpu.sync_copy(y_hbm.at[d_v.at[3, k]], z_v.at[6 + k])
                pltpu.sync_copy(z_v.at[6 + k], z_hbm.at[k, pl.ds(base, G), :])

        for h in writes:
            h.wait()

    return s2(dest, y)


TK = 28
NKC = T // TK
TB = TK * B


def _cd_body(z_ref, w_ref, w2_ref, b2_ref, w3_ref, b3_ref,
             w4_ref, b4_ref, w5_ref, b5_ref, w6_ref, b6_ref,
             o_ref, acc_ref):
    i = pl.program_id(0)

    @pl.when(i == 0)
    def _():
        acc_ref[...] = jnp.zeros_like(acc_ref)

    w = w_ref[...]
    a = w[:, 0:1] * z_ref[0] + w[:, 1:2] * z_ref[1]
    a = jnp.maximum(a, 0.0).astype(jnp.bfloat16)
    y = jnp.dot(a, w2_ref[...], preferred_element_type=F32) + b2_ref[...]
    y = jnp.maximum(y, 0.0).astype(jnp.bfloat16)
    y = jnp.dot(y, w3_ref[...], preferred_element_type=F32) + b3_ref[...]
    y = jnp.maximum(y, 0.0).astype(jnp.bfloat16)
    y = y.reshape(TK, B, H)
    acc = acc_ref[...]
    for tk in range(TK):
        acc += jnp.dot(y[tk], w4_ref[tk].astype(jnp.bfloat16),
                       preferred_element_type=F32)
    acc_ref[...] = acc

    @pl.when(i == NKC - 1)
    def _():
        zz = jnp.maximum(acc_ref[...] + b4_ref[...], 0.0).astype(jnp.bfloat16)
        zz = jnp.dot(zz, w5_ref[...].astype(jnp.bfloat16),
                     preferred_element_type=F32) + b5_ref[...]
        zz = jnp.maximum(zz, 0.0).astype(jnp.bfloat16)
        o_ref[...] = jnp.dot(zz, w6_ref[...].astype(jnp.bfloat16),
                             preferred_element_type=F32) + b6_ref[...]


def _cd(z, wco, fc2_w, fc2_b, fc3_w, fc3_b, fc4_w, fc4_b,
        fc5_w, fc5_b, fc6_w, fc6_b):
    return pl.pallas_call(
        _cd_body,
        grid=(NKC,),
        in_specs=[
            pl.BlockSpec((2, TB, H), lambda i: (0, i, 0)),
            pl.BlockSpec((TB, 2), lambda i: (i, 0)),
            pl.BlockSpec((H, H), lambda i: (0, 0)),
            pl.BlockSpec((1, H), lambda i: (0, 0)),
            pl.BlockSpec((H, H), lambda i: (0, 0)),
            pl.BlockSpec((1, H), lambda i: (0, 0)),
            pl.BlockSpec((TK, H, H), lambda i: (i, 0, 0)),
            pl.BlockSpec((1, H), lambda i: (0, 0)),
            pl.BlockSpec((H, H), lambda i: (0, 0)),
            pl.BlockSpec((1, H), lambda i: (0, 0)),
            pl.BlockSpec((H, OUT), lambda i: (0, 0)),
            pl.BlockSpec((1, OUT), lambda i: (0, 0)),
        ],
        out_specs=pl.BlockSpec((B, OUT), lambda i: (0, 0)),
        out_shape=jax.ShapeDtypeStruct((B, OUT), F32),
        scratch_shapes=[pltpu.VMEM((B, H), F32)],
        compiler_params=pltpu.CompilerParams(
            dimension_semantics=("arbitrary",)),
    )(z, wco, fc2_w.astype(jnp.bfloat16), fc2_b.reshape(1, H),
      fc3_w.astype(jnp.bfloat16), fc3_b.reshape(1, H),
      fc4_w.reshape(T, H, H), fc4_b.reshape(1, H),
      fc5_w, fc5_b.reshape(1, H), fc6_w, fc6_b.reshape(1, OUT))


def kernel(x, gate_w, gate_b, ew1, eb1, ew2, eb2, ew3, eb3,
           fc2_w, fc2_b, fc3_w, fc3_b, fc4_w, fc4_b,
           fc5_w, fc5_b, fc6_w, fc6_b):
    x2d = jnp.transpose(x, (1, 0, 2)).reshape(N, D)
    code, wco, po2, eot_pad = _gate(x2d, gate_w, gate_b)
    xs, dest = _dispatch(x2d, code, po2)
    y = _moe_mlp(eot_pad, xs, ew1, eb1, ew2, eb2, ew3, eb3)
    z = _gather_pairs(dest, y)
    return _cd(z, wco, fc2_w, fc2_b, fc3_w, fc3_b, fc4_w, fc4_b,
               fc5_w, fc5_b, fc6_w, fc6_b)


# speedup vs baseline: 1.5095x; 1.0674x over previous
import dataclasses
import functools

import jax
import jax.numpy as jnp
from jax import lax
from jax.experimental import pallas as pl
from jax.experimental.pallas import tpu as pltpu
from jax.experimental.pallas import tpu_sc as plsc

B, T, D, E, K, H, OUT = 32, 196, 768, 16, 2, 128, 18
N = B * T
TILE = 128
NT = N // TILE
GT = 448
NGT = N // GT
RCAP = (N * K // TILE + E) * TILE
NRT = RCAP // TILE
G = 64
NG = N // G
NSUB = 32
F32 = jnp.float32
I32 = jnp.int32

_SC_PARAMS = pltpu.CompilerParams()
if "needs_layout_passes" in pltpu.CompilerParams.__dataclass_fields__:
    _SC_PARAMS = dataclasses.replace(_SC_PARAMS, needs_layout_passes=False)


def _gate_body(x_ref, gw_ref, gb_ref, code_ref, w_ref,
               po_ref, eot_ref, c0_ref, c1_ref):
    i = pl.program_id(0)

    @pl.when(i == 0)
    def _():
        c0_ref[...] = jnp.zeros_like(c0_ref)
        c1_ref[...] = jnp.zeros_like(c1_ref)

    x = x_ref[...]
    logits = jnp.dot(x, gw_ref[...], preferred_element_type=F32)
    logits = logits + gb_ref[...]
    m = jnp.max(logits, axis=1, keepdims=True)
    ex = jnp.exp(logits - m)
    probs = ex / jnp.sum(ex, axis=1, keepdims=True)

    idx = lax.broadcasted_iota(I32, (GT, E), 1)
    m0 = jnp.max(probs, axis=1, keepdims=True)
    e0 = jnp.min(jnp.where(probs >= m0, idx, E + 1), axis=1)
    oh0 = (idx == e0[:, None]).astype(F32)
    probs2 = jnp.where(oh0 > 0, -1.0, probs)
    m1 = jnp.max(probs2, axis=1, keepdims=True)
    e1 = jnp.min(jnp.where(probs2 >= m1, idx, E + 1), axis=1)
    oh1 = (idx == e1[:, None]).astype(F32)

    r = lax.broadcasted_iota(I32, (GT, GT), 0)
    c = lax.broadcasted_iota(I32, (GT, GT), 1)
    lt = (r > c).astype(F32)
    ex0 = jnp.dot(lt, oh0, preferred_element_type=F32)
    ex1 = jnp.dot(lt, oh1, preferred_element_type=F32)
    rank0 = jnp.sum(ex0 * oh0, axis=1) + jnp.sum(oh0 * c0_ref[...], axis=1)
    rank1 = jnp.sum(ex1 * oh1, axis=1) + jnp.sum(oh1 * c1_ref[...], axis=1)
    c0_ref[...] = c0_ref[...] + jnp.sum(oh0, axis=0, keepdims=True)
    c1_ref[...] = c1_ref[...] + jnp.sum(oh1, axis=0, keepdims=True)

    col = lax.broadcasted_iota(I32, (GT, 2), 1)
    code0 = (rank0.astype(I32) << 4) | e0
    code1 = (rank1.astype(I32) << 4) | e1
    code_ref[...] = jnp.where(col == 0, code0[:, None], code1[:, None])
    w_ref[...] = jnp.where(col == 0, probs[:, 0:1], probs[:, 1:2])

    @pl.when(i == NGT - 1)
    def _():
        h0 = c0_ref[...]
        ht = c0_ref[...] + c1_ref[...]
        padded = jnp.ceil(ht / TILE) * TILE
        rr = lax.broadcasted_iota(I32, (E, E), 0)
        cc = lax.broadcasted_iota(I32, (E, E), 1)
        su = (rr < cc).astype(F32)
        po = jnp.dot(padded, su, preferred_element_type=F32)
        prow = lax.broadcasted_iota(I32, (8, E), 0)
        po_ref[...] = jnp.where(prow == 1, po + h0, po).astype(I32)
        tpos = lax.broadcasted_iota(I32, (TILE, E), 0).astype(F32) * TILE
        eot = jnp.sum((tpos >= po).astype(I32), axis=1) - 1
        eot = jnp.clip(eot, 0, E - 1)
        eot_ref[...] = jnp.broadcast_to(eot[None, :], (8, TILE))


def _gate(x2d, gate_w, gate_b):
    return pl.pallas_call(
        _gate_body,
        grid=(NGT,),
        in_specs=[
            pl.BlockSpec((GT, D), lambda i: (i, 0)),
            pl.BlockSpec((D, E), lambda i: (0, 0)),
            pl.BlockSpec((1, E), lambda i: (0, 0)),
        ],
        out_specs=[
            pl.BlockSpec((GT, 2), lambda i: (i, 0)),
            pl.BlockSpec((GT, 2), lambda i: (i, 0)),
            pl.BlockSpec((8, E), lambda i: (0, 0)),
            pl.BlockSpec((8, TILE), lambda i: (0, 0)),
        ],
        out_shape=[
            jax.ShapeDtypeStruct((N, 2), I32),
            jax.ShapeDtypeStruct((N, 2), F32),
            jax.ShapeDtypeStruct((8, E), I32),
            jax.ShapeDtypeStruct((8, TILE), I32),
        ],
        scratch_shapes=[pltpu.VMEM((1, E), F32), pltpu.VMEM((1, E), F32)],
        compiler_params=pltpu.CompilerParams(
            dimension_semantics=("arbitrary",)),
    )(x2d, gate_w, gate_b.reshape(1, E))


DI = D // 2
def _dispatch(xb, code, po2):
    @functools.partial(
        pl.kernel,
        mesh=plsc.VectorSubcoreMesh(core_axis_name="c", subcore_axis_name="s"),
        out_type=[jax.ShapeDtypeStruct((RCAP, D), F32),
                  jax.ShapeDtypeStruct((2, N), I32)],
        scratch_types=[pltpu.VMEM((2, G, D), F32),
                       pltpu.VMEM((G, 2), I32),
                       pltpu.VMEM((4, 2, G), I32),
                       pltpu.VMEM((2, E), I32),
                       pltpu.SemaphoreType.DMA,
                       pltpu.SemaphoreType.DMA,
                       pltpu.SemaphoreType.DMA,
                       pltpu.SemaphoreType.DMA],
        compiler_params=_SC_PARAMS,
    )
    def s1(x_hbm, code_hbm, po_hbm, xs_hbm, dest_hbm,
           rows_v, cd_v, dscr_v, po_v, sl0, sl1, ss0, ss1):
        wid = lax.axis_index("s") * 2 + lax.axis_index("c")
        base0 = wid * G
        base1 = (wid + NSUB) * G
        base2 = (wid + 2 * NSUB) * G
        ld0 = pltpu.make_async_copy(x_hbm.at[pl.ds(base0, G), :],
                                    rows_v.at[0], sl0)
        ld0.start()
        ld1 = pltpu.make_async_copy(x_hbm.at[pl.ds(base1, G), :],
                                    rows_v.at[1], sl1)
        ld1.start()
        pltpu.sync_copy(po_hbm.at[pl.ds(0, 2), :], po_v)

        def compute_dest(base, j):
            pltpu.sync_copy(code_hbm.at[pl.ds(base, G), :], cd_v)
            for k in range(2):
                kf = jnp.full((16,), k, I32)
                for jj in range(G // 16):
                    ridx = lax.iota(I32, 16) + 16 * jj
                    cd = plsc.load_gather(cd_v, [ridx, kf])
                    e = jnp.bitwise_and(cd, E - 1)
                    rk = lax.shift_right_logical(cd, 4)
                    dscr_v[j, k, pl.ds(16 * jj, 16)] = (
                        plsc.load_gather(po_v, [kf, e]) + rk)
            for k in range(2):
                pltpu.sync_copy(dscr_v.at[j, k], dest_hbm.at[k, pl.ds(base, G)])

        compute_dest(base0, 0)
        compute_dest(base1, 1)
        compute_dest(base2, 2)

        def scatter(j, buf, sem):
            hs = [pltpu.make_async_copy(rows_v.at[buf],
                                        xs_hbm.at[dscr_v.at[j, k]], sem)
                  for k in range(2)]
            for h in hs:
                h.start()
            return hs

        ld0.wait()
        sc0 = scatter(0, 0, ss0)
        ld1.wait()
        sc1 = scatter(1, 1, ss1)
        for h in sc0:
            h.wait()
        ld2 = pltpu.make_async_copy(x_hbm.at[pl.ds(base2, G), :],
                                    rows_v.at[0], sl0)
        ld2.start()
        ld2.wait()
        sc2 = scatter(2, 0, ss0)
        for h in sc1:
            h.wait()

        @pl.when(wid < NG - 3 * NSUB)
        def _():
            base3 = (wid + 3 * NSUB) * G
            pltpu.sync_copy(x_hbm.at[pl.ds(base3, G), :], rows_v.at[1])
            compute_dest(base3, 3)
            for k in range(2):
                pltpu.sync_copy(rows_v.at[1], xs_hbm.at[dscr_v.at[3, k]])

        for h in sc2:
            h.wait()

    return s1(xb, code, po2)


def _mlp_body(eot_ref, xs_ref, w1_ref, b1_ref, w2_ref, b2_ref, w3_ref, b3_ref,
              y_ref):
    x = xs_ref[...].astype(jnp.bfloat16)
    h = jnp.dot(x, w1_ref[0], preferred_element_type=F32) + b1_ref[0]
    h = jnp.maximum(h, 0.0).astype(jnp.bfloat16)
    h = jnp.dot(h, w2_ref[0], preferred_element_type=F32) + b2_ref[0]
    h = jnp.maximum(h, 0.0).astype(jnp.bfloat16)
    h = jnp.dot(h, w3_ref[0], preferred_element_type=F32) + b3_ref[0]
    y_ref[...] = jnp.maximum(h, 0.0)


def _moe_mlp(eot, xs, ew1, eb1, ew2, eb2, ew3, eb3):
    grid_spec = pltpu.PrefetchScalarGridSpec(
        num_scalar_prefetch=1,
        grid=(NRT,),
        in_specs=[
            pl.BlockSpec((TILE, D), lambda i, eot: (i, 0)),
            pl.BlockSpec((1, D, H), lambda i, eot: (eot[0, i], 0, 0)),
            pl.BlockSpec((1, 1, H), lambda i, eot: (eot[0, i], 0, 0)),
            pl.BlockSpec((1, H, H), lambda i, eot: (eot[0, i], 0, 0)),
            pl.BlockSpec((1, 1, H), lambda i, eot: (eot[0, i], 0, 0)),
            pl.BlockSpec((1, H, H), lambda i, eot: (eot[0, i], 0, 0)),
            pl.BlockSpec((1, 1, H), lambda i, eot: (eot[0, i], 0, 0)),
        ],
        out_specs=pl.BlockSpec((TILE, H), lambda i, eot: (i, 0)),
    )
    return pl.pallas_call(
        _mlp_body,
        grid_spec=grid_spec,
        out_shape=jax.ShapeDtypeStruct((RCAP, H), F32),
        compiler_params=pltpu.CompilerParams(
            dimension_semantics=("arbitrary",)),
    )(eot, xs, ew1.astype(jnp.bfloat16), eb1.reshape(E, 1, H),
      ew2.astype(jnp.bfloat16), eb2.reshape(E, 1, H),
      ew3.astype(jnp.bfloat16), eb3.reshape(E, 1, H))


def _gather_pairs(dest, y):
    @functools.partial(
        pl.kernel,
        mesh=plsc.VectorSubcoreMesh(core_axis_name="c", subcore_axis_name="s"),
        out_type=jax.ShapeDtypeStruct((2, N, H), F32),
        scratch_types=[pltpu.VMEM((8, G, H), F32),
                       pltpu.VMEM((4, 2, G), I32),
                       pltpu.SemaphoreType.DMA,
                       pltpu.SemaphoreType.DMA],
        compiler_params=_SC_PARAMS,
    )
    def s2(dest_hbm, y_hbm, z_hbm, z_v, d_v, sg, sw):
        wid = lax.axis_index("s") * 2 + lax.axis_index("c")
        gathers = []
        for j in range(3):
            base = (wid + NSUB * j) * G
            for k in range(2):
                pltpu.sync_copy(dest_hbm.at[k, pl.ds(base, G)], d_v.at[j, k])
                gathers.append(pltpu.make_async_copy(
                    y_hbm.at[d_v.at[j, k]], z_v.at[2 * j + k], sg))
                gathers[-1].start()
        for h in gathers:
            h.wait()
        writes = []
        for j in range(3):
            base = (wid + NSUB * j) * G
            for k in range(2):
                writes.append(pltpu.make_async_copy(
                    z_v.at[2 * j + k], z_hbm.at[k, pl.ds(base, G), :], sw))
                writes[-1].start()

        @pl.when(wid < NG - 3 * NSUB)
        def _():
            base = (wid + 3 * NSUB) * G
            for k in range(2):
                pltpu.sync_copy(dest_hbm.at[k, pl.ds(base, G)], d_v.at[3, k])
                pltpu.sync_copy(y_hbm.at[d_v.at[3, k]], z_v.at[6 + k])
                pltpu.sync_copy(z_v.at[6 + k], z_hbm.at[k, pl.ds(base, G), :])

        for h in writes:
            h.wait()

    return s2(dest, y)


TK = 28
NKC = T // TK
TB = TK * B


def _cd_body(z_ref, w_ref, w2_ref, b2_ref, w3_ref, b3_ref,
             w4_ref, b4_ref, w5_ref, b5_ref, w6_ref, b6_ref,
             o_ref, acc_ref):
    i = pl.program_id(0)

    @pl.when(i == 0)
    def _():
        acc_ref[...] = jnp.zeros_like(acc_ref)

    w = w_ref[...]
    a = w[:, 0:1] * z_ref[0] + w[:, 1:2] * z_ref[1]
    a = jnp.maximum(a, 0.0).astype(jnp.bfloat16)
    y = jnp.dot(a, w2_ref[...], preferred_element_type=F32) + b2_ref[...]
    y = jnp.maximum(y, 0.0).astype(jnp.bfloat16)
    y = jnp.dot(y, w3_ref[...], preferred_element_type=F32) + b3_ref[...]
    y = jnp.maximum(y, 0.0).astype(jnp.bfloat16)
    y = y.reshape(TK, B, H)
    acc = acc_ref[...]
    for tk in range(TK):
        acc += jnp.dot(y[tk], w4_ref[tk].astype(jnp.bfloat16),
                       preferred_element_type=F32)
    acc_ref[...] = acc

    @pl.when(i == NKC - 1)
    def _():
        zz = jnp.maximum(acc_ref[...] + b4_ref[...], 0.0).astype(jnp.bfloat16)
        zz = jnp.dot(zz, w5_ref[...].astype(jnp.bfloat16),
                     preferred_element_type=F32) + b5_ref[...]
        zz = jnp.maximum(zz, 0.0).astype(jnp.bfloat16)
        o_ref[...] = jnp.dot(zz, w6_ref[...].astype(jnp.bfloat16),
                             preferred_element_type=F32) + b6_ref[...]


def _cd(z, wco, fc2_w, fc2_b, fc3_w, fc3_b, fc4_w, fc4_b,
        fc5_w, fc5_b, fc6_w, fc6_b):
    return pl.pallas_call(
        _cd_body,
        grid=(NKC,),
        in_specs=[
            pl.BlockSpec((2, TB, H), lambda i: (0, i, 0)),
            pl.BlockSpec((TB, 2), lambda i: (i, 0)),
            pl.BlockSpec((H, H), lambda i: (0, 0)),
            pl.BlockSpec((1, H), lambda i: (0, 0)),
            pl.BlockSpec((H, H), lambda i: (0, 0)),
            pl.BlockSpec((1, H), lambda i: (0, 0)),
            pl.BlockSpec((TK, H, H), lambda i: (i, 0, 0)),
            pl.BlockSpec((1, H), lambda i: (0, 0)),
            pl.BlockSpec((H, H), lambda i: (0, 0)),
            pl.BlockSpec((1, H), lambda i: (0, 0)),
            pl.BlockSpec((H, OUT), lambda i: (0, 0)),
            pl.BlockSpec((1, OUT), lambda i: (0, 0)),
        ],
        out_specs=pl.BlockSpec((B, OUT), lambda i: (0, 0)),
        out_shape=jax.ShapeDtypeStruct((B, OUT), F32),
        scratch_shapes=[pltpu.VMEM((B, H), F32)],
        compiler_params=pltpu.CompilerParams(
            dimension_semantics=("arbitrary",)),
    )(z, wco, fc2_w.astype(jnp.bfloat16), fc2_b.reshape(1, H),
      fc3_w.astype(jnp.bfloat16), fc3_b.reshape(1, H),
      fc4_w.reshape(T, H, H), fc4_b.reshape(1, H),
      fc5_w, fc5_b.reshape(1, H), fc6_w, fc6_b.reshape(1, OUT))


def kernel(x, gate_w, gate_b, ew1, eb1, ew2, eb2, ew3, eb3,
           fc2_w, fc2_b, fc3_w, fc3_b, fc4_w, fc4_b,
           fc5_w, fc5_b, fc6_w, fc6_b):
    x2d = jnp.transpose(x, (1, 0, 2)).reshape(N, D)
    code, wco, po2, eot_pad = _gate(x2d, gate_w, gate_b)
    xs, dest = _dispatch(x2d, code, po2)
    y = _moe_mlp(eot_pad, xs, ew1, eb1, ew2, eb2, ew3, eb3)
    z = _gather_pairs(dest, y)
    return _cd(z, wco, fc2_w, fc2_b, fc3_w, fc3_b, fc4_w, fc4_b,
               fc5_w, fc5_b, fc6_w, fc6_b)


# speedup vs baseline: 1.5409x; 1.0208x over previous
import dataclasses
import functools

import jax
import jax.numpy as jnp
from jax import lax
from jax.experimental import pallas as pl
from jax.experimental.pallas import tpu as pltpu
from jax.experimental.pallas import tpu_sc as plsc

B, T, D, E, K, H, OUT = 32, 196, 768, 16, 2, 128, 18
N = B * T
TILE = 128
NT = N // TILE
GT = 896
NGT = N // GT
RCAP = (N * K // TILE + E) * TILE
NRT = RCAP // TILE
G = 64
NG = N // G
NSUB = 32
F32 = jnp.float32
I32 = jnp.int32

_SC_PARAMS = pltpu.CompilerParams()
if "needs_layout_passes" in pltpu.CompilerParams.__dataclass_fields__:
    _SC_PARAMS = dataclasses.replace(_SC_PARAMS, needs_layout_passes=False)


def _gate_body(x_ref, gw_ref, gb_ref, code_ref, w_ref,
               po_ref, eot_ref, c0_ref, c1_ref):
    i = pl.program_id(0)

    @pl.when(i == 0)
    def _():
        c0_ref[...] = jnp.zeros_like(c0_ref)
        c1_ref[...] = jnp.zeros_like(c1_ref)

    x = x_ref[...]
    logits = jnp.dot(x, gw_ref[...], preferred_element_type=F32)
    logits = logits + gb_ref[...]
    m = jnp.max(logits, axis=1, keepdims=True)
    ex = jnp.exp(logits - m)
    probs = ex / jnp.sum(ex, axis=1, keepdims=True)

    idx = lax.broadcasted_iota(I32, (GT, E), 1)
    m0 = jnp.max(probs, axis=1, keepdims=True)
    e0 = jnp.min(jnp.where(probs >= m0, idx, E + 1), axis=1)
    oh0 = (idx == e0[:, None]).astype(F32)
    probs2 = jnp.where(oh0 > 0, -1.0, probs)
    m1 = jnp.max(probs2, axis=1, keepdims=True)
    e1 = jnp.min(jnp.where(probs2 >= m1, idx, E + 1), axis=1)
    oh1 = (idx == e1[:, None]).astype(F32)

    r = lax.broadcasted_iota(I32, (GT, GT), 0)
    c = lax.broadcasted_iota(I32, (GT, GT), 1)
    lt = (r > c).astype(F32)
    ex0 = jnp.dot(lt, oh0, preferred_element_type=F32)
    ex1 = jnp.dot(lt, oh1, preferred_element_type=F32)
    rank0 = jnp.sum(ex0 * oh0, axis=1) + jnp.sum(oh0 * c0_ref[...], axis=1)
    rank1 = jnp.sum(ex1 * oh1, axis=1) + jnp.sum(oh1 * c1_ref[...], axis=1)
    c0_ref[...] = c0_ref[...] + jnp.sum(oh0, axis=0, keepdims=True)
    c1_ref[...] = c1_ref[...] + jnp.sum(oh1, axis=0, keepdims=True)

    col = lax.broadcasted_iota(I32, (GT, 2), 1)
    code0 = (rank0.astype(I32) << 4) | e0
    code1 = (rank1.astype(I32) << 4) | e1
    code_ref[...] = jnp.where(col == 0, code0[:, None], code1[:, None])
    w_ref[...] = jnp.where(col == 0, probs[:, 0:1], probs[:, 1:2])

    @pl.when(i == NGT - 1)
    def _():
        h0 = c0_ref[...]
        ht = c0_ref[...] + c1_ref[...]
        padded = jnp.ceil(ht / TILE) * TILE
        rr = lax.broadcasted_iota(I32, (E, E), 0)
        cc = lax.broadcasted_iota(I32, (E, E), 1)
        su = (rr < cc).astype(F32)
        po = jnp.dot(padded, su, preferred_element_type=F32)
        prow = lax.broadcasted_iota(I32, (8, E), 0)
        po_ref[...] = jnp.where(prow == 1, po + h0, po).astype(I32)
        tpos = lax.broadcasted_iota(I32, (TILE, E), 0).astype(F32) * TILE
        eot = jnp.sum((tpos >= po).astype(I32), axis=1) - 1
        eot = jnp.clip(eot, 0, E - 1)
        eot_ref[...] = jnp.broadcast_to(eot[None, :], (8, TILE))


def _gate(x2d, gate_w, gate_b):
    return pl.pallas_call(
        _gate_body,
        grid=(NGT,),
        in_specs=[
            pl.BlockSpec((GT, D), lambda i: (i, 0)),
            pl.BlockSpec((D, E), lambda i: (0, 0)),
            pl.BlockSpec((1, E), lambda i: (0, 0)),
        ],
        out_specs=[
            pl.BlockSpec((GT, 2), lambda i: (i, 0)),
            pl.BlockSpec((GT, 2), lambda i: (i, 0)),
            pl.BlockSpec((8, E), lambda i: (0, 0)),
            pl.BlockSpec((8, TILE), lambda i: (0, 0)),
        ],
        out_shape=[
            jax.ShapeDtypeStruct((N, 2), I32),
            jax.ShapeDtypeStruct((N, 2), F32),
            jax.ShapeDtypeStruct((8, E), I32),
            jax.ShapeDtypeStruct((8, TILE), I32),
        ],
        scratch_shapes=[pltpu.VMEM((1, E), F32), pltpu.VMEM((1, E), F32)],
        compiler_params=pltpu.CompilerParams(
            dimension_semantics=("arbitrary",)),
    )(x2d, gate_w, gate_b.reshape(1, E))


DI = D // 2
def _dispatch(xb, code, po2):
    @functools.partial(
        pl.kernel,
        mesh=plsc.VectorSubcoreMesh(core_axis_name="c", subcore_axis_name="s"),
        out_type=[jax.ShapeDtypeStruct((RCAP, D), F32),
                  jax.ShapeDtypeStruct((2, N), I32)],
        scratch_types=[pltpu.VMEM((2, G, D), F32),
                       pltpu.VMEM((G, 2), I32),
                       pltpu.VMEM((4, 2, G), I32),
                       pltpu.VMEM((2, E), I32),
                       pltpu.SemaphoreType.DMA,
                       pltpu.SemaphoreType.DMA,
                       pltpu.SemaphoreType.DMA,
                       pltpu.SemaphoreType.DMA],
        compiler_params=_SC_PARAMS,
    )
    def s1(x_hbm, code_hbm, po_hbm, xs_hbm, dest_hbm,
           rows_v, cd_v, dscr_v, po_v, sl0, sl1, ss0, ss1):
        wid = lax.axis_index("s") * 2 + lax.axis_index("c")
        base0 = wid * G
        base1 = (wid + NSUB) * G
        base2 = (wid + 2 * NSUB) * G
        ld0 = pltpu.make_async_copy(x_hbm.at[pl.ds(base0, G), :],
                                    rows_v.at[0], sl0)
        ld0.start()
        ld1 = pltpu.make_async_copy(x_hbm.at[pl.ds(base1, G), :],
                                    rows_v.at[1], sl1)
        ld1.start()
        pltpu.sync_copy(po_hbm.at[pl.ds(0, 2), :], po_v)

        def compute_dest(base, j):
            pltpu.sync_copy(code_hbm.at[pl.ds(base, G), :], cd_v)
            for k in range(2):
                kf = jnp.full((16,), k, I32)
                for jj in range(G // 16):
                    ridx = lax.iota(I32, 16) + 16 * jj
                    cd = plsc.load_gather(cd_v, [ridx, kf])
                    e = jnp.bitwise_and(cd, E - 1)
                    rk = lax.shift_right_logical(cd, 4)
                    dscr_v[j, k, pl.ds(16 * jj, 16)] = (
                        plsc.load_gather(po_v, [kf, e]) + rk)
            for k in range(2):
                pltpu.sync_copy(dscr_v.at[j, k], dest_hbm.at[k, pl.ds(base, G)])

        compute_dest(base0, 0)
        compute_dest(base1, 1)
        compute_dest(base2, 2)

        def scatter(j, buf, sem):
            hs = [pltpu.make_async_copy(rows_v.at[buf],
                                        xs_hbm.at[dscr_v.at[j, k]], sem)
                  for k in range(2)]
            for h in hs:
                h.start()
            return hs

        ld0.wait()
        sc0 = scatter(0, 0, ss0)
        ld1.wait()
        sc1 = scatter(1, 1, ss1)
        for h in sc0:
            h.wait()
        ld2 = pltpu.make_async_copy(x_hbm.at[pl.ds(base2, G), :],
                                    rows_v.at[0], sl0)
        ld2.start()
        ld2.wait()
        sc2 = scatter(2, 0, ss0)
        for h in sc1:
            h.wait()

        @pl.when(wid < NG - 3 * NSUB)
        def _():
            base3 = (wid + 3 * NSUB) * G
            pltpu.sync_copy(x_hbm.at[pl.ds(base3, G), :], rows_v.at[1])
            compute_dest(base3, 3)
            for k in range(2):
                pltpu.sync_copy(rows_v.at[1], xs_hbm.at[dscr_v.at[3, k]])

        for h in sc2:
            h.wait()

    return s1(xb, code, po2)


def _mlp_body(eot_ref, xs_ref, w1_ref, b1_ref, w2_ref, b2_ref, w3_ref, b3_ref,
              y_ref):
    x = xs_ref[...].astype(jnp.bfloat16)
    h = jnp.dot(x, w1_ref[0], preferred_element_type=F32) + b1_ref[0]
    h = jnp.maximum(h, 0.0).astype(jnp.bfloat16)
    h = jnp.dot(h, w2_ref[0], preferred_element_type=F32) + b2_ref[0]
    h = jnp.maximum(h, 0.0).astype(jnp.bfloat16)
    h = jnp.dot(h, w3_ref[0], preferred_element_type=F32) + b3_ref[0]
    y_ref[...] = jnp.maximum(h, 0.0)


def _moe_mlp(eot, xs, ew1, eb1, ew2, eb2, ew3, eb3):
    grid_spec = pltpu.PrefetchScalarGridSpec(
        num_scalar_prefetch=1,
        grid=(NRT,),
        in_specs=[
            pl.BlockSpec((TILE, D), lambda i, eot: (i, 0)),
            pl.BlockSpec((1, D, H), lambda i, eot: (eot[0, i], 0, 0)),
            pl.BlockSpec((1, 1, H), lambda i, eot: (eot[0, i], 0, 0)),
            pl.BlockSpec((1, H, H), lambda i, eot: (eot[0, i], 0, 0)),
            pl.BlockSpec((1, 1, H), lambda i, eot: (eot[0, i], 0, 0)),
            pl.BlockSpec((1, H, H), lambda i, eot: (eot[0, i], 0, 0)),
            pl.BlockSpec((1, 1, H), lambda i, eot: (eot[0, i], 0, 0)),
        ],
        out_specs=pl.BlockSpec((TILE, H), lambda i, eot: (i, 0)),
    )
    return pl.pallas_call(
        _mlp_body,
        grid_spec=grid_spec,
        out_shape=jax.ShapeDtypeStruct((RCAP, H), F32),
        compiler_params=pltpu.CompilerParams(
            dimension_semantics=("arbitrary",)),
    )(eot, xs, ew1.astype(jnp.bfloat16), eb1.reshape(E, 1, H),
      ew2.astype(jnp.bfloat16), eb2.reshape(E, 1, H),
      ew3.astype(jnp.bfloat16), eb3.reshape(E, 1, H))


def _gather_pairs(dest, y):
    @functools.partial(
        pl.kernel,
        mesh=plsc.VectorSubcoreMesh(core_axis_name="c", subcore_axis_name="s"),
        out_type=jax.ShapeDtypeStruct((2, N, H), F32),
        scratch_types=[pltpu.VMEM((8, G, H), F32),
                       pltpu.VMEM((4, 2, G), I32),
                       pltpu.SemaphoreType.DMA,
                       pltpu.SemaphoreType.DMA],
        compiler_params=_SC_PARAMS,
    )
    def s2(dest_hbm, y_hbm, z_hbm, z_v, d_v, sg, sw):
        wid = lax.axis_index("s") * 2 + lax.axis_index("c")
        gathers = []
        for j in range(3):
            base = (wid + NSUB * j) * G
            for k in range(2):
                pltpu.sync_copy(dest_hbm.at[k, pl.ds(base, G)], d_v.at[j, k])
                gathers.append(pltpu.make_async_copy(
                    y_hbm.at[d_v.at[j, k]], z_v.at[2 * j + k], sg))
                gathers[-1].start()
        for h in gathers:
            h.wait()
        writes = []
        for j in range(3):
            base = (wid + NSUB * j) * G
            for k in range(2):
                writes.append(pltpu.make_async_copy(
                    z_v.at[2 * j + k], z_hbm.at[k, pl.ds(base, G), :], sw))
                writes[-1].start()

        @pl.when(wid < NG - 3 * NSUB)
        def _():
            base = (wid + 3 * NSUB) * G
            for k in range(2):
                pltpu.sync_copy(dest_hbm.at[k, pl.ds(base, G)], d_v.at[3, k])
                pltpu.sync_copy(y_hbm.at[d_v.at[3, k]], z_v.at[6 + k])
                pltpu.sync_copy(z_v.at[6 + k], z_hbm.at[k, pl.ds(base, G), :])

        for h in writes:
            h.wait()

    return s2(dest, y)


TK = 28
NKC = T // TK
TB = TK * B


def _cd_body(z_ref, w_ref, w2_ref, b2_ref, w3_ref, b3_ref,
             w4_ref, b4_ref, w5_ref, b5_ref, w6_ref, b6_ref,
             o_ref, acc_ref):
    i = pl.program_id(0)

    @pl.when(i == 0)
    def _():
        acc_ref[...] = jnp.zeros_like(acc_ref)

    w = w_ref[...]
    a = w[:, 0:1] * z_ref[0] + w[:, 1:2] * z_ref[1]
    a = jnp.maximum(a, 0.0).astype(jnp.bfloat16)
    y = jnp.dot(a, w2_ref[...], preferred_element_type=F32) + b2_ref[...]
    y = jnp.maximum(y, 0.0).astype(jnp.bfloat16)
    y = jnp.dot(y, w3_ref[...], preferred_element_type=F32) + b3_ref[...]
    y = jnp.maximum(y, 0.0).astype(jnp.bfloat16)
    y = y.reshape(TK, B, H)
    acc = acc_ref[...]
    for tk in range(TK):
        acc += jnp.dot(y[tk], w4_ref[tk].astype(jnp.bfloat16),
                       preferred_element_type=F32)
    acc_ref[...] = acc

    @pl.when(i == NKC - 1)
    def _():
        zz = jnp.maximum(acc_ref[...] + b4_ref[...], 0.0).astype(jnp.bfloat16)
        zz = jnp.dot(zz, w5_ref[...].astype(jnp.bfloat16),
                     preferred_element_type=F32) + b5_ref[...]
        zz = jnp.maximum(zz, 0.0).astype(jnp.bfloat16)
        o_ref[...] = jnp.dot(zz, w6_ref[...].astype(jnp.bfloat16),
                             preferred_element_type=F32) + b6_ref[...]


def _cd(z, wco, fc2_w, fc2_b, fc3_w, fc3_b, fc4_w, fc4_b,
        fc5_w, fc5_b, fc6_w, fc6_b):
    return pl.pallas_call(
        _cd_body,
        grid=(NKC,),
        in_specs=[
            pl.BlockSpec((2, TB, H), lambda i: (0, i, 0)),
            pl.BlockSpec((TB, 2), lambda i: (i, 0)),
            pl.BlockSpec((H, H), lambda i: (0, 0)),
            pl.BlockSpec((1, H), lambda i: (0, 0)),
            pl.BlockSpec((H, H), lambda i: (0, 0)),
            pl.BlockSpec((1, H), lambda i: (0, 0)),
            pl.BlockSpec((TK, H, H), lambda i: (i, 0, 0)),
            pl.BlockSpec((1, H), lambda i: (0, 0)),
            pl.BlockSpec((H, H), lambda i: (0, 0)),
            pl.BlockSpec((1, H), lambda i: (0, 0)),
            pl.BlockSpec((H, OUT), lambda i: (0, 0)),
            pl.BlockSpec((1, OUT), lambda i: (0, 0)),
        ],
        out_specs=pl.BlockSpec((B, OUT), lambda i: (0, 0)),
        out_shape=jax.ShapeDtypeStruct((B, OUT), F32),
        scratch_shapes=[pltpu.VMEM((B, H), F32)],
        compiler_params=pltpu.CompilerParams(
            dimension_semantics=("arbitrary",)),
    )(z, wco, fc2_w.astype(jnp.bfloat16), fc2_b.reshape(1, H),
      fc3_w.astype(jnp.bfloat16), fc3_b.reshape(1, H),
      fc4_w.reshape(T, H, H), fc4_b.reshape(1, H),
      fc5_w, fc5_b.reshape(1, H), fc6_w, fc6_b.reshape(1, OUT))


def kernel(x, gate_w, gate_b, ew1, eb1, ew2, eb2, ew3, eb3,
           fc2_w, fc2_b, fc3_w, fc3_b, fc4_w, fc4_b,
           fc5_w, fc5_b, fc6_w, fc6_b):
    x2d = jnp.transpose(x, (1, 0, 2)).reshape(N, D)
    code, wco, po2, eot_pad = _gate(x2d, gate_w, gate_b)
    xs, dest = _dispatch(x2d, code, po2)
    y = _moe_mlp(eot_pad, xs, ew1, eb1, ew2, eb2, ew3, eb3)
    z = _gather_pairs(dest, y)
    return _cd(z, wco, fc2_w, fc2_b, fc3_w, fc3_b, fc4_w, fc4_b,
               fc5_w, fc5_b, fc6_w, fc6_b)
